# Initial kernel scaffold; baseline (speedup 1.0000x reference)
import functools

import jax
import jax.numpy as jnp
from jax import lax
from jax.experimental import pallas as pl
from jax.experimental.pallas import tpu as pltpu
from jax.experimental.pallas import tpu_sc as plsc

N = 10000
E = 320000
D = 128
HEADS, HC, OUT, CLS_HID = 8, 16, 64, 16

NP = 10016
EP = 330240
NC, NS = 2, 16
NW = NC * NS
EW = EP // NW
G = 120
CHUNKS = EW // G
RPS = NP // NS

_BN_SCALE = 1.0 / (1.0 + 1e-5) ** 0.5

f32 = jnp.float32
i32 = jnp.int32


def _sc_gat_kernel(heads, hc):
    hwc = heads * hc
    aw = hwc + 16
    nv = hwc // 16
    vph = hc // 16
    mesh = plsc.VectorSubcoreMesh(core_axis_name="c", subcore_axis_name="s")

    @functools.partial(
        pl.kernel,
        out_type=jax.ShapeDtypeStruct((NC, NP, aw), f32),
        mesh=mesh,
        scratch_types=[
            pltpu.VMEM((G,), i32),
            pltpu.VMEM((G,), i32),
            pltpu.VMEM((G, hwc), f32),
            pltpu.VMEM((G, hwc), f32),
            pltpu.VMEM((G, aw), f32),
            pltpu.VMEM((hwc,), f32),
            pltpu.VMEM_SHARED((NP, aw), f32),
            pltpu.SemaphoreType.DMA,
            pltpu.SemaphoreType.DMA,
        ],
    )
    def kern(xl_hbm, xr_hbm, src_hbm, dst_hbm, att_hbm, out_hbm,
             idxs, idxd, bufl, bufr, val, att_v, acc, sem1, sem2):
        c = lax.axis_index("c")
        s = lax.axis_index("s")
        wid = c * NS + s
        lanes = lax.iota(i32, 16)
        zero16 = jnp.zeros((16,), f32)

        pltpu.sync_copy(att_hbm, att_v)
        att_regs = [att_v[pl.ds(16 * j, 16)] for j in range(nv)]

        def zrow(e, _):
            for j in range(aw // 16):
                val[e, pl.ds(16 * j, 16)] = zero16
            return 0

        lax.fori_loop(0, G, zrow, 0)
        row0 = s * RPS
        off = 0
        for sz in (G,) * (RPS // G) + ((RPS % G,) if RPS % G else ()):
            pltpu.sync_copy(val.at[pl.ds(0, sz)], acc.at[pl.ds(row0 + off, sz)])
            off += sz
        plsc.subcore_barrier()

        def edge_body(e, _):
            wsel = zero16
            for h in range(heads):
                a = f32(0.0)
                for v in range(vph):
                    j = h * vph + v
                    z = bufl[e, pl.ds(16 * j, 16)] + bufr[e, pl.ds(16 * j, 16)]
                    z = jnp.maximum(z, 0.2 * z)
                    a = a + jnp.sum(z * att_regs[j])
                wb = jnp.exp(jnp.minimum(jnp.full((16,), a, f32), 60.0))
                for v in range(vph):
                    j = h * vph + v
                    val[e, pl.ds(16 * j, 16)] = bufl[e, pl.ds(16 * j, 16)] * wb
                wsel = jnp.where(lanes == h, wb, wsel)
            val[e, pl.ds(hwc, 16)] = wsel
            return 0

        base = wid * EW

        def chunk(k, _):
            o = base + k * G
            pltpu.sync_copy(src_hbm.at[pl.ds(o, G)], idxs)
            pltpu.sync_copy(dst_hbm.at[pl.ds(o, G)], idxd)
            cpl = pltpu.async_copy(xl_hbm.at[idxs], bufl, sem1)
            cpr = pltpu.async_copy(xr_hbm.at[idxd], bufr, sem2)
            cpl.wait()
            cpr.wait()
            lax.fori_loop(0, G, edge_body, 0)
            pltpu.sync_copy(val, acc.at[idxd], add=True)
            return 0

        lax.fori_loop(0, CHUNKS, chunk, 0)
        plsc.subcore_barrier()

        off = 0
        for sz in (G,) * (RPS // G) + ((RPS % G,) if RPS % G else ()):
            pltpu.sync_copy(acc.at[pl.ds(row0 + off, sz)],
                            out_hbm.at[c, pl.ds(row0 + off, sz)])
            off += sz

    return kern


_sc_gat_128 = _sc_gat_kernel(HEADS, HC)
_sc_gat_64 = _sc_gat_kernel(1, OUT)


def _tc_call(body, out_shapes, *args):
    return pl.pallas_call(body, out_shape=out_shapes)(*args)


def _tc_pre(x, g0, b0, wl, bl, wr, br):
    def body(x_ref, g_ref, b_ref, wl_ref, bl_ref, wr_ref, br_ref,
             h_ref, xl_ref, xr_ref):
        h = x_ref[...] * (g_ref[...] * _BN_SCALE) + b_ref[...]
        h_ref[...] = h
        xl_ref[...] = jnp.dot(h, wl_ref[...], preferred_element_type=f32) + bl_ref[...]
        xr_ref[...] = jnp.dot(h, wr_ref[...], preferred_element_type=f32) + br_ref[...]

    outs = [jax.ShapeDtypeStruct((NP, D), f32)] * 3
    return _tc_call(body, outs, x, g0, b0, wl, bl, wr, br)


def _tc_mid(acc, bias, g, b, hprev, wl, bl, wr, br, dout):
    hwc = HEADS * HC

    def body(a_ref, bias_ref, g_ref, b_ref, hp_ref, wl_ref, bl_ref,
             wr_ref, br_ref, h_ref, xl_ref, xr_ref):
        a = a_ref[0] + a_ref[1]
        num = a[:, :hwc]
        den = a[:, hwc:hwc + HEADS]
        hh = lax.broadcasted_iota(i32, (HEADS, hwc), 0)
        cc = lax.broadcasted_iota(i32, (HEADS, hwc), 1)
        rep = jnp.where(cc // HC == hh, f32(1.0), f32(0.0))
        den_rep = jnp.dot(den, rep, preferred_element_type=f32)
        o = num / (den_rep + 1e-16) + bias_ref[...]
        o = o * (g_ref[...] * _BN_SCALE) + b_ref[...]
        o = jnp.where(o > 0, o, jnp.expm1(o))
        h = o + hp_ref[...]
        h_ref[...] = h
        xl_ref[...] = jnp.dot(h, wl_ref[...], preferred_element_type=f32) + bl_ref[...]
        xr_ref[...] = jnp.dot(h, wr_ref[...], preferred_element_type=f32) + br_ref[...]

    outs = [jax.ShapeDtypeStruct((NP, hwc), f32),
            jax.ShapeDtypeStruct((NP, dout), f32),
            jax.ShapeDtypeStruct((NP, dout), f32)]
    return _tc_call(body, outs, acc, bias, g, b, hprev, wl, bl, wr, br)


def _tc_post(acc, bias, wc1, bc1, wc2, bc2):
    def body(a_ref, bias_ref, w1_ref, b1_ref, w2_ref, b2_ref, y_ref):
        a = a_ref[0] + a_ref[1]
        num = a[:, :OUT]
        den = a[:, OUT:OUT + 1]
        ones = jnp.full((1, OUT), f32(1.0))
        den_rep = jnp.dot(den, ones, preferred_element_type=f32)
        o = num / (den_rep + 1e-16) + bias_ref[...]
        y = jnp.dot(o, w1_ref[...], preferred_element_type=f32) + b1_ref[...]
        y = jnp.maximum(y, 0.0)
        y_ref[...] = jnp.dot(y, w2_ref[...], preferred_element_type=f32) + b2_ref[...]

    outs = jax.ShapeDtypeStruct((NP, OUT), f32)
    return _tc_call(body, outs, acc, bias, wc1, bc1, wc2, bc2)


def kernel(x, edge_index, params):
    p = params
    src = edge_index[0].astype(i32)
    dst = edge_index[1].astype(i32)
    dstm = jnp.where(src == dst, N, dst)
    loops = jnp.arange(N, dtype=i32)
    padi = jnp.full((EP - E - N,), N, i32)
    src_e = jnp.concatenate([src, loops, padi])
    dst_e = jnp.concatenate([dstm, loops, padi])

    xp = jnp.pad(x.astype(f32), ((0, NP - N), (0, 0)))

    def row(v):
        return v.reshape(1, -1).astype(f32)

    c1 = p['conv1']
    h0, xl, xr = _tc_pre(xp, row(p['g0']), row(p['b0']),
                         c1['Wl'], row(c1['bl']), c1['Wr'], row(c1['br']))
    acc1 = _sc_gat_128(xl, xr, src_e, dst_e, c1['att'].reshape(-1))

    c2 = p['conv2']
    h1, xl, xr = _tc_mid(acc1, row(c1['bias']), row(p['g1']), row(p['b1']),
                         h0, c2['Wl'], row(c2['bl']), c2['Wr'], row(c2['br']), D)
    acc2 = _sc_gat_128(xl, xr, src_e, dst_e, c2['att'].reshape(-1))

    c3 = p['conv3']
    h2, xl, xr = _tc_mid(acc2, row(c2['bias']), row(p['g2']), row(p['b2']),
                         h1, c3['Wl'], row(c3['bl']), c3['Wr'], row(c3['br']), OUT)
    acc3 = _sc_gat_64(xl, xr, src_e, dst_e, c3['att'].reshape(-1))

    y = _tc_post(acc3, row(c3['bias']), p['Wc1'], row(p['bc1']),
                 p['Wc2'], row(p['bc2']))
    return y[:N]


# speedup vs baseline: 12.9915x; 12.9915x over previous
import functools

import jax
import jax.numpy as jnp
from jax import lax
from jax.experimental import pallas as pl
from jax.experimental.pallas import tpu as pltpu
from jax.experimental.pallas import tpu_sc as plsc

N = 10000
E = 320000
D = 128
HEADS, HC, OUT, CLS_HID = 8, 16, 64, 16

NP = 10112
EP = 330240
NC, NS = 2, 16
NW = NC * NS
EW = EP // NW
G = 80
CHUNKS = EW // G
RPS = NP // NS

_BN_SCALE = 1.0 / (1.0 + 1e-5) ** 0.5

f32 = jnp.float32
i32 = jnp.int32


@functools.cache
def _sc_gat_kernel(heads, hc):
    hwc = heads * hc
    aw = hwc + 16
    nv = hwc // 16
    vph = hc // 16
    mesh = plsc.VectorSubcoreMesh(core_axis_name="c", subcore_axis_name="s",
                                  num_cores=NC, num_subcores=NS)

    @functools.partial(
        pl.kernel,
        out_type=jax.ShapeDtypeStruct((NC, NP, aw), f32),
        mesh=mesh,
        scratch_types=[
            pltpu.VMEM((G,), i32),
            pltpu.VMEM((G,), i32),
            pltpu.VMEM((G, hwc), f32),
            pltpu.VMEM((G, hwc), f32),
            pltpu.VMEM((G, aw), f32),
            pltpu.VMEM((hwc,), f32),
            pltpu.VMEM_SHARED((NP, aw), f32),
            pltpu.SemaphoreType.DMA,
            pltpu.SemaphoreType.DMA,
        ],
        compiler_params=pltpu.CompilerParams(needs_layout_passes=False,
                                             use_tc_tiling_on_sc=False),
    )
    def kern(xl_hbm, xr_hbm, src_hbm, dst_hbm, att_hbm, out_hbm,
             idxs, idxd, bufl, bufr, val, att_v, acc, sem1, sem2):
        c = lax.axis_index("c")
        s = lax.axis_index("s")
        wid = c * NS + s
        lanes = lax.iota(i32, 16)
        zero16 = jnp.zeros((16,), f32)

        pltpu.sync_copy(att_hbm, att_v)
        att_regs = [att_v[pl.ds(16 * j, 16)] for j in range(nv)]

        def zrow(e, _):
            for j in range(aw // 16):
                val[e, pl.ds(16 * j, 16)] = zero16
            return 0

        lax.fori_loop(i32(0), i32(G), zrow, 0)
        row0 = s * i32(RPS)
        off = 0
        for sz in (G,) * (RPS // G) + ((RPS % G,) if RPS % G else ()):
            pltpu.sync_copy(val.at[pl.ds(0, sz)], acc.at[pl.ds(row0 + off, sz)])
            off += sz
        plsc.subcore_barrier()

        def edge_body(e, _):
            wsel = zero16
            for h in range(heads):
                a = f32(0.0)
                for v in range(vph):
                    j = h * vph + v
                    z = bufl[e, pl.ds(16 * j, 16)] + bufr[e, pl.ds(16 * j, 16)]
                    z = jnp.maximum(z, 0.2 * z)
                    a = a + jnp.sum(z * att_regs[j])
                wb = jnp.exp(jnp.minimum(jnp.full((16,), a, f32), 60.0))
                for v in range(vph):
                    j = h * vph + v
                    val[e, pl.ds(16 * j, 16)] = bufl[e, pl.ds(16 * j, 16)] * wb
                wsel = jnp.where(lanes == h, wb, wsel)
            val[e, pl.ds(hwc, 16)] = wsel
            return 0

        base = wid * i32(EW)

        def chunk(k, _):
            o = base + k * i32(G)
            pltpu.sync_copy(src_hbm.at[pl.ds(o, G)], idxs)
            pltpu.sync_copy(dst_hbm.at[pl.ds(o, G)], idxd)
            cpl = pltpu.async_copy(xl_hbm.at[idxs], bufl, sem1)
            cpr = pltpu.async_copy(xr_hbm.at[idxd], bufr, sem2)
            cpl.wait()
            cpr.wait()
            lax.fori_loop(i32(0), i32(G), edge_body, 0)
            pltpu.sync_copy(val, acc.at[idxd], add=True)
            return 0

        lax.fori_loop(i32(0), i32(CHUNKS), chunk, 0)
        plsc.subcore_barrier()

        off = 0
        for sz in (G,) * (RPS // G) + ((RPS % G,) if RPS % G else ()):
            pltpu.sync_copy(acc.at[pl.ds(row0 + off, sz)],
                            out_hbm.at[c, pl.ds(row0 + off, sz)])
            off += sz

    return kern


def _tc_call(body, out_shapes, *args):
    return pl.pallas_call(body, out_shape=out_shapes)(*args)


def _tc_pre(x, g0, b0, wl, bl, wr, br):
    def body(x_ref, g_ref, b_ref, wl_ref, bl_ref, wr_ref, br_ref,
             h_ref, xl_ref, xr_ref):
        h = x_ref[...] * (g_ref[...] * _BN_SCALE) + b_ref[...]
        h_ref[...] = h
        xl_ref[...] = jnp.dot(h, wl_ref[...], preferred_element_type=f32) + bl_ref[...]
        xr_ref[...] = jnp.dot(h, wr_ref[...], preferred_element_type=f32) + br_ref[...]

    outs = [jax.ShapeDtypeStruct((NP, D), f32)] * 3
    return _tc_call(body, outs, x, g0, b0, wl, bl, wr, br)


def _div_den(a, heads, hc):
    hwc = heads * hc
    num = a[:, :hwc]
    den = a[:, hwc:hwc + heads]
    hh = lax.broadcasted_iota(i32, (heads, hwc), 0)
    cc = lax.broadcasted_iota(i32, (heads, hwc), 1)
    rep = jnp.where(cc // hc == hh, f32(1.0), f32(0.0))
    den_rep = jnp.dot(den, rep, preferred_element_type=f32)
    return num / (den_rep + 1e-16)


def _tc_mid(acc, bias, g, b, hprev, wl, bl, wr, br, dout):
    hwc = HEADS * HC

    def body(a_ref, bias_ref, g_ref, b_ref, hp_ref, wl_ref, bl_ref,
             wr_ref, br_ref, h_ref, xl_ref, xr_ref):
        a = a_ref[0] + a_ref[1]
        o = _div_den(a, HEADS, HC) + bias_ref[...]
        o = o * (g_ref[...] * _BN_SCALE) + b_ref[...]
        o = jnp.where(o > 0, o, jnp.exp(o) - 1.0)
        h = o + hp_ref[...]
        h_ref[...] = h
        xl_ref[...] = jnp.dot(h, wl_ref[...], preferred_element_type=f32) + bl_ref[...]
        xr_ref[...] = jnp.dot(h, wr_ref[...], preferred_element_type=f32) + br_ref[...]

    outs = [jax.ShapeDtypeStruct((NP, hwc), f32),
            jax.ShapeDtypeStruct((NP, dout), f32),
            jax.ShapeDtypeStruct((NP, dout), f32)]
    return _tc_call(body, outs, acc, bias, g, b, hprev, wl, bl, wr, br)


def _tc_post(acc, bias, wc1, bc1, wc2, bc2):
    def body(a_ref, bias_ref, w1_ref, b1_ref, w2_ref, b2_ref, y_ref):
        a = a_ref[0] + a_ref[1]
        o = _div_den(a, 1, OUT) + bias_ref[...]
        y = jnp.dot(o, w1_ref[...], preferred_element_type=f32) + b1_ref[...]
        y = jnp.maximum(y, 0.0)
        y_ref[...] = jnp.dot(y, w2_ref[...], preferred_element_type=f32) + b2_ref[...]

    outs = jax.ShapeDtypeStruct((NP, OUT), f32)
    return _tc_call(body, outs, acc, bias, wc1, bc1, wc2, bc2)


def kernel(x, edge_index, params):
    p = params
    src = edge_index[0].astype(i32)
    dst = edge_index[1].astype(i32)
    dstm = jnp.where(src == dst, N, dst)
    loops = jnp.arange(N, dtype=i32)
    padi = jnp.full((EP - E - N,), N, i32)
    src_e = jnp.concatenate([src, loops, padi])
    dst_e = jnp.concatenate([dstm, loops, padi])

    xp = jnp.pad(x.astype(f32), ((0, NP - N), (0, 0)))

    def row(v):
        return v.reshape(1, -1).astype(f32)

    c1 = p['conv1']
    h0, xl, xr = _tc_pre(xp, row(p['g0']), row(p['b0']),
                         c1['Wl'], row(c1['bl']), c1['Wr'], row(c1['br']))
    acc1 = _sc_gat_kernel(HEADS, HC)(xl, xr, src_e, dst_e, c1['att'].reshape(-1))

    c2 = p['conv2']
    h1, xl, xr = _tc_mid(acc1, row(c1['bias']), row(p['g1']), row(p['b1']),
                         h0, c2['Wl'], row(c2['bl']), c2['Wr'], row(c2['br']), D)
    acc2 = _sc_gat_kernel(HEADS, HC)(xl, xr, src_e, dst_e, c2['att'].reshape(-1))

    c3 = p['conv3']
    h2, xl, xr = _tc_mid(acc2, row(c2['bias']), row(p['g2']), row(p['b2']),
                         h1, c3['Wl'], row(c3['bl']), c3['Wr'], row(c3['br']), OUT)
    acc3 = _sc_gat_kernel(1, OUT)(xl, xr, src_e, dst_e, c3['att'].reshape(-1))

    y = _tc_post(acc3, row(c3['bias']), p['Wc1'], row(p['bc1']),
                 p['Wc2'], row(p['bc2']))
    return y[:N]


# speedup vs baseline: 40.2878x; 3.1011x over previous
import functools

import jax
import jax.numpy as jnp
from jax import lax
from jax.experimental import pallas as pl
from jax.experimental.pallas import tpu as pltpu
from jax.experimental.pallas import tpu_sc as plsc

N = 10000
E = 320000
D = 128
HEADS, HC, OUT, CLS_HID = 8, 16, 64, 16

NP = 10112
EP = 330240
NC, NS = 2, 16
NW = NC * NS
EW = EP // NW
G = 80
CHUNKS = EW // G
RPS = NP // NS

_BN_SCALE = 1.0 / (1.0 + 1e-5) ** 0.5

f32 = jnp.float32
i32 = jnp.int32


@functools.cache
def _sc_gat_kernel(heads, hc):
    hwc = heads * hc
    aw = hwc + 16
    nv = hwc // 16
    vph = hc // 16
    mesh = plsc.VectorSubcoreMesh(core_axis_name="c", subcore_axis_name="s",
                                  num_cores=NC, num_subcores=NS)

    @functools.partial(
        pl.kernel,
        out_type=jax.ShapeDtypeStruct((NC, NP, aw), f32),
        mesh=mesh,
        scratch_types=[
            pltpu.VMEM((G,), i32),
            pltpu.VMEM((G,), i32),
            pltpu.VMEM((G, hwc), f32),
            pltpu.VMEM((G, hwc), f32),
            pltpu.VMEM((G, aw), f32),
            pltpu.VMEM((hwc,), f32),
            pltpu.VMEM_SHARED((NP, aw), f32),
            pltpu.SemaphoreType.DMA,
            pltpu.SemaphoreType.DMA,
        ],
        compiler_params=pltpu.CompilerParams(needs_layout_passes=False,
                                             use_tc_tiling_on_sc=False),
    )
    def kern(xl_hbm, xr_hbm, src_hbm, dst_hbm, att_hbm, out_hbm,
             idxs, idxd, bufl, bufr, val, att_v, acc, sem1, sem2):
        c = lax.axis_index("c")
        s = lax.axis_index("s")
        wid = c * NS + s
        lanes = lax.iota(i32, 16)
        zero16 = jnp.zeros((16,), f32)

        pltpu.sync_copy(att_hbm, att_v)
        att_regs = [att_v[pl.ds(16 * j, 16)] for j in range(nv)]

        def zrow(e, _):
            for j in range(aw // 16):
                val[e, pl.ds(16 * j, 16)] = zero16
            return 0

        lax.fori_loop(i32(0), i32(G), zrow, 0)
        row0 = s * i32(RPS)
        off = 0
        for sz in (G,) * (RPS // G) + ((RPS % G,) if RPS % G else ()):
            pltpu.sync_copy(val.at[pl.ds(0, sz)], acc.at[pl.ds(row0 + off, sz)])
            off += sz
        plsc.subcore_barrier()

        lane15 = jnp.full((16, 1), 15, i32)
        _gdn = lax.GatherDimensionNumbers(
            offset_dims=(), collapsed_slice_dims=(0,), start_index_map=(0,))

        def _bcast(v, idx_vec):
            return lax.gather(v, idx_vec, _gdn, (1,),
                              mode=lax.GatherScatterMode.PROMISE_IN_BOUNDS)

        def edge_body(e, _):
            zls = []
            ts = []
            for h in range(heads):
                acc_t = None
                for v in range(vph):
                    j = h * vph + v
                    zl = bufl[e, pl.ds(16 * j, 16)]
                    zls.append(zl)
                    z = zl + bufr[e, pl.ds(16 * j, 16)]
                    z = jnp.maximum(z, 0.2 * z)
                    t = z * att_regs[j]
                    acc_t = t if acc_t is None else acc_t + t
                ts.append(acc_t)
            tots = [_bcast(plsc.cumsum(t), lane15) for t in ts]
            alpha = jnp.full((16,), -100.0, f32)
            for h in range(heads):
                alpha = jnp.where(lanes == h, tots[h], alpha)
            wv = jnp.exp(jnp.minimum(alpha, 60.0))
            for h in range(heads):
                wb = _bcast(wv, jnp.full((16, 1), h, i32))
                for v in range(vph):
                    j = h * vph + v
                    val[e, pl.ds(16 * j, 16)] = zls[j] * wb
            val[e, pl.ds(hwc, 16)] = wv
            return 0

        base = wid * i32(EW)

        def chunk(k, _):
            o = base + k * i32(G)
            pltpu.sync_copy(src_hbm.at[pl.ds(o, G)], idxs)
            pltpu.sync_copy(dst_hbm.at[pl.ds(o, G)], idxd)
            cpl = pltpu.async_copy(xl_hbm.at[idxs], bufl, sem1)
            cpr = pltpu.async_copy(xr_hbm.at[idxd], bufr, sem2)
            cpl.wait()
            cpr.wait()
            lax.fori_loop(i32(0), i32(G), edge_body, 0)
            pltpu.sync_copy(val, acc.at[idxd], add=True)
            return 0

        lax.fori_loop(i32(0), i32(CHUNKS), chunk, 0)
        plsc.subcore_barrier()

        off = 0
        for sz in (G,) * (RPS // G) + ((RPS % G,) if RPS % G else ()):
            pltpu.sync_copy(acc.at[pl.ds(row0 + off, sz)],
                            out_hbm.at[c, pl.ds(row0 + off, sz)])
            off += sz

    return kern


def _tc_call(body, out_shapes, *args):
    return pl.pallas_call(body, out_shape=out_shapes)(*args)


def _tc_pre(x, g0, b0, wl, bl, wr, br):
    def body(x_ref, g_ref, b_ref, wl_ref, bl_ref, wr_ref, br_ref,
             h_ref, xl_ref, xr_ref):
        h = x_ref[...] * (g_ref[...] * _BN_SCALE) + b_ref[...]
        h_ref[...] = h
        xl_ref[...] = jnp.dot(h, wl_ref[...], preferred_element_type=f32) + bl_ref[...]
        xr_ref[...] = jnp.dot(h, wr_ref[...], preferred_element_type=f32) + br_ref[...]

    outs = [jax.ShapeDtypeStruct((NP, D), f32)] * 3
    return _tc_call(body, outs, x, g0, b0, wl, bl, wr, br)


def _div_den(a, heads, hc):
    hwc = heads * hc
    num = a[:, :hwc]
    den = a[:, hwc:hwc + heads]
    hh = lax.broadcasted_iota(i32, (heads, hwc), 0)
    cc = lax.broadcasted_iota(i32, (heads, hwc), 1)
    rep = jnp.where(cc // hc == hh, f32(1.0), f32(0.0))
    den_rep = jnp.dot(den, rep, preferred_element_type=f32)
    return num / (den_rep + 1e-16)


def _tc_mid(acc, bias, g, b, hprev, wl, bl, wr, br, dout):
    hwc = HEADS * HC

    def body(a_ref, bias_ref, g_ref, b_ref, hp_ref, wl_ref, bl_ref,
             wr_ref, br_ref, h_ref, xl_ref, xr_ref):
        a = a_ref[0] + a_ref[1]
        o = _div_den(a, HEADS, HC) + bias_ref[...]
        o = o * (g_ref[...] * _BN_SCALE) + b_ref[...]
        o = jnp.where(o > 0, o, jnp.exp(o) - 1.0)
        h = o + hp_ref[...]
        h_ref[...] = h
        xl_ref[...] = jnp.dot(h, wl_ref[...], preferred_element_type=f32) + bl_ref[...]
        xr_ref[...] = jnp.dot(h, wr_ref[...], preferred_element_type=f32) + br_ref[...]

    outs = [jax.ShapeDtypeStruct((NP, hwc), f32),
            jax.ShapeDtypeStruct((NP, dout), f32),
            jax.ShapeDtypeStruct((NP, dout), f32)]
    return _tc_call(body, outs, acc, bias, g, b, hprev, wl, bl, wr, br)


def _tc_post(acc, bias, wc1, bc1, wc2, bc2):
    def body(a_ref, bias_ref, w1_ref, b1_ref, w2_ref, b2_ref, y_ref):
        a = a_ref[0] + a_ref[1]
        o = _div_den(a, 1, OUT) + bias_ref[...]
        y = jnp.dot(o, w1_ref[...], preferred_element_type=f32) + b1_ref[...]
        y = jnp.maximum(y, 0.0)
        y_ref[...] = jnp.dot(y, w2_ref[...], preferred_element_type=f32) + b2_ref[...]

    outs = jax.ShapeDtypeStruct((NP, OUT), f32)
    return _tc_call(body, outs, acc, bias, wc1, bc1, wc2, bc2)


def kernel(x, edge_index, params):
    p = params
    src = edge_index[0].astype(i32)
    dst = edge_index[1].astype(i32)
    dstm = jnp.where(src == dst, N, dst)
    loops = jnp.arange(N, dtype=i32)
    padi = jnp.full((EP - E - N,), N, i32)
    src_e = jnp.concatenate([src, loops, padi])
    dst_e = jnp.concatenate([dstm, loops, padi])

    xp = jnp.pad(x.astype(f32), ((0, NP - N), (0, 0)))

    def row(v):
        return v.reshape(1, -1).astype(f32)

    c1 = p['conv1']
    h0, xl, xr = _tc_pre(xp, row(p['g0']), row(p['b0']),
                         c1['Wl'], row(c1['bl']), c1['Wr'], row(c1['br']))
    acc1 = _sc_gat_kernel(HEADS, HC)(xl, xr, src_e, dst_e, c1['att'].reshape(-1))

    c2 = p['conv2']
    h1, xl, xr = _tc_mid(acc1, row(c1['bias']), row(p['g1']), row(p['b1']),
                         h0, c2['Wl'], row(c2['bl']), c2['Wr'], row(c2['br']), D)
    acc2 = _sc_gat_kernel(HEADS, HC)(xl, xr, src_e, dst_e, c2['att'].reshape(-1))

    c3 = p['conv3']
    h2, xl, xr = _tc_mid(acc2, row(c2['bias']), row(p['g2']), row(p['b2']),
                         h1, c3['Wl'], row(c3['bl']), c3['Wr'], row(c3['br']), OUT)
    acc3 = _sc_gat_kernel(1, OUT)(xl, xr, src_e, dst_e, c3['att'].reshape(-1))

    y = _tc_post(acc3, row(c3['bias']), p['Wc1'], row(p['bc1']),
                 p['Wc2'], row(p['bc2']))
    return y[:N]


# speedup vs baseline: 41.7807x; 1.0371x over previous
import functools

import jax
import jax.numpy as jnp
from jax import lax
from jax.experimental import pallas as pl
from jax.experimental.pallas import tpu as pltpu
from jax.experimental.pallas import tpu_sc as plsc

N = 10000
E = 320000
D = 128
HEADS, HC, OUT, CLS_HID = 8, 16, 64, 16

NP = 10112
EP = 330240
NC, NS = 2, 16
NW = NC * NS
EW = EP // NW
G = 80
CHUNKS = EW // G
RPS = NP // NS

_BN_SCALE = 1.0 / (1.0 + 1e-5) ** 0.5

f32 = jnp.float32
i32 = jnp.int32


@functools.cache
def _sc_gat_kernel(heads, hc):
    hwc = heads * hc
    aw = hwc + 16
    nv = hwc // 16
    vph = hc // 16
    mesh = plsc.VectorSubcoreMesh(core_axis_name="c", subcore_axis_name="s",
                                  num_cores=NC, num_subcores=NS)

    @functools.partial(
        pl.kernel,
        out_type=jax.ShapeDtypeStruct((NC, NP, aw), f32),
        mesh=mesh,
        scratch_types=[
            pltpu.VMEM((G,), i32),
            pltpu.VMEM((G,), i32),
            pltpu.VMEM((G, hwc), f32),
            pltpu.VMEM((G, hwc), f32),
            pltpu.VMEM((G, aw), f32),
            pltpu.VMEM((hwc,), f32),
            pltpu.VMEM_SHARED((NP, aw), f32),
            pltpu.SemaphoreType.DMA,
            pltpu.SemaphoreType.DMA,
        ],
        compiler_params=pltpu.CompilerParams(needs_layout_passes=False,
                                             use_tc_tiling_on_sc=False),
    )
    def kern(xl_hbm, xr_hbm, src_hbm, dst_hbm, att_hbm, out_hbm,
             idxs, idxd, bufl, bufr, val, att_v, acc, sem1, sem2):
        c = lax.axis_index("c")
        s = lax.axis_index("s")
        wid = c * NS + s
        lanes = lax.iota(i32, 16)
        zero16 = jnp.zeros((16,), f32)

        pltpu.sync_copy(att_hbm, att_v)
        att_regs = [att_v[pl.ds(16 * j, 16)] for j in range(nv)]

        def zrow(e, _):
            for j in range(aw // 16):
                val[e, pl.ds(16 * j, 16)] = zero16
            return 0

        lax.fori_loop(i32(0), i32(G), zrow, 0)
        row0 = s * i32(RPS)
        off = 0
        for sz in (G,) * (RPS // G) + ((RPS % G,) if RPS % G else ()):
            pltpu.sync_copy(val.at[pl.ds(0, sz)], acc.at[pl.ds(row0 + off, sz)])
            off += sz
        plsc.subcore_barrier()

        lane15 = jnp.full((16, 1), 15, i32)
        _gdn = lax.GatherDimensionNumbers(
            offset_dims=(), collapsed_slice_dims=(0,), start_index_map=(0,))

        def _bcast(v, idx_vec):
            return lax.gather(v, idx_vec, _gdn, (1,),
                              mode=lax.GatherScatterMode.PROMISE_IN_BOUNDS)

        def edge_body(e):
            zls = []
            ts = []
            for h in range(heads):
                acc_t = None
                for v in range(vph):
                    j = h * vph + v
                    zl = bufl[e, pl.ds(16 * j, 16)]
                    zls.append(zl)
                    z = zl + bufr[e, pl.ds(16 * j, 16)]
                    z = jnp.maximum(z, 0.2 * z)
                    t = z * att_regs[j]
                    acc_t = t if acc_t is None else acc_t + t
                ts.append(acc_t)
            tots = [_bcast(plsc.cumsum(t), lane15) for t in ts]
            alpha = jnp.full((16,), -100.0, f32)
            for h in range(heads):
                alpha = jnp.where(lanes == h, tots[h], alpha)
            wv = jnp.exp(jnp.minimum(alpha, 60.0))
            for h in range(heads):
                wb = _bcast(wv, jnp.full((16, 1), h, i32))
                for v in range(vph):
                    j = h * vph + v
                    val[e, pl.ds(16 * j, 16)] = zls[j] * wb
            val[e, pl.ds(hwc, 16)] = wv

        base = wid * i32(EW)

        def chunk(k, _):
            o = base + k * i32(G)
            pltpu.sync_copy(src_hbm.at[pl.ds(o, G)], idxs)
            pltpu.sync_copy(dst_hbm.at[pl.ds(o, G)], idxd)
            cpl = pltpu.async_copy(xl_hbm.at[idxs], bufl, sem1)
            cpr = pltpu.async_copy(xr_hbm.at[idxd], bufr, sem2)
            cpl.wait()
            cpr.wait()
            def edge_pair(i, car):
                edge_body(2 * i)
                edge_body(2 * i + 1)
                return car

            lax.fori_loop(i32(0), i32(G // 2), edge_pair, 0)
            pltpu.sync_copy(val, acc.at[idxd], add=True)
            return 0

        lax.fori_loop(i32(0), i32(CHUNKS), chunk, 0)
        plsc.subcore_barrier()

        off = 0
        for sz in (G,) * (RPS // G) + ((RPS % G,) if RPS % G else ()):
            pltpu.sync_copy(acc.at[pl.ds(row0 + off, sz)],
                            out_hbm.at[c, pl.ds(row0 + off, sz)])
            off += sz

    return kern


def _tc_call(body, out_shapes, *args):
    return pl.pallas_call(body, out_shape=out_shapes)(*args)


def _tc_pre(x, g0, b0, wl, bl, wr, br):
    def body(x_ref, g_ref, b_ref, wl_ref, bl_ref, wr_ref, br_ref,
             h_ref, xl_ref, xr_ref):
        h = x_ref[...] * (g_ref[...] * _BN_SCALE) + b_ref[...]
        h_ref[...] = h
        xl_ref[...] = jnp.dot(h, wl_ref[...], preferred_element_type=f32) + bl_ref[...]
        xr_ref[...] = jnp.dot(h, wr_ref[...], preferred_element_type=f32) + br_ref[...]

    outs = [jax.ShapeDtypeStruct((NP, D), f32)] * 3
    return _tc_call(body, outs, x, g0, b0, wl, bl, wr, br)


def _div_den(a, heads, hc):
    hwc = heads * hc
    num = a[:, :hwc]
    den = a[:, hwc:hwc + heads]
    hh = lax.broadcasted_iota(i32, (heads, hwc), 0)
    cc = lax.broadcasted_iota(i32, (heads, hwc), 1)
    rep = jnp.where(cc // hc == hh, f32(1.0), f32(0.0))
    den_rep = jnp.dot(den, rep, preferred_element_type=f32)
    return num / (den_rep + 1e-16)


def _tc_mid(acc, bias, g, b, hprev, wl, bl, wr, br, dout):
    hwc = HEADS * HC

    def body(a_ref, bias_ref, g_ref, b_ref, hp_ref, wl_ref, bl_ref,
             wr_ref, br_ref, h_ref, xl_ref, xr_ref):
        a = a_ref[0] + a_ref[1]
        o = _div_den(a, HEADS, HC) + bias_ref[...]
        o = o * (g_ref[...] * _BN_SCALE) + b_ref[...]
        o = jnp.where(o > 0, o, jnp.exp(o) - 1.0)
        h = o + hp_ref[...]
        h_ref[...] = h
        xl_ref[...] = jnp.dot(h, wl_ref[...], preferred_element_type=f32) + bl_ref[...]
        xr_ref[...] = jnp.dot(h, wr_ref[...], preferred_element_type=f32) + br_ref[...]

    outs = [jax.ShapeDtypeStruct((NP, hwc), f32),
            jax.ShapeDtypeStruct((NP, dout), f32),
            jax.ShapeDtypeStruct((NP, dout), f32)]
    return _tc_call(body, outs, acc, bias, g, b, hprev, wl, bl, wr, br)


def _tc_post(acc, bias, wc1, bc1, wc2, bc2):
    def body(a_ref, bias_ref, w1_ref, b1_ref, w2_ref, b2_ref, y_ref):
        a = a_ref[0] + a_ref[1]
        o = _div_den(a, 1, OUT) + bias_ref[...]
        y = jnp.dot(o, w1_ref[...], preferred_element_type=f32) + b1_ref[...]
        y = jnp.maximum(y, 0.0)
        y_ref[...] = jnp.dot(y, w2_ref[...], preferred_element_type=f32) + b2_ref[...]

    outs = jax.ShapeDtypeStruct((NP, OUT), f32)
    return _tc_call(body, outs, acc, bias, wc1, bc1, wc2, bc2)


def kernel(x, edge_index, params):
    p = params
    src = edge_index[0].astype(i32)
    dst = edge_index[1].astype(i32)
    dstm = jnp.where(src == dst, N, dst)
    loops = jnp.arange(N, dtype=i32)
    padi = jnp.full((EP - E - N,), N, i32)
    src_e = jnp.concatenate([src, loops, padi])
    dst_e = jnp.concatenate([dstm, loops, padi])

    xp = jnp.pad(x.astype(f32), ((0, NP - N), (0, 0)))

    def row(v):
        return v.reshape(1, -1).astype(f32)

    c1 = p['conv1']
    h0, xl, xr = _tc_pre(xp, row(p['g0']), row(p['b0']),
                         c1['Wl'], row(c1['bl']), c1['Wr'], row(c1['br']))
    acc1 = _sc_gat_kernel(HEADS, HC)(xl, xr, src_e, dst_e, c1['att'].reshape(-1))

    c2 = p['conv2']
    h1, xl, xr = _tc_mid(acc1, row(c1['bias']), row(p['g1']), row(p['b1']),
                         h0, c2['Wl'], row(c2['bl']), c2['Wr'], row(c2['br']), D)
    acc2 = _sc_gat_kernel(HEADS, HC)(xl, xr, src_e, dst_e, c2['att'].reshape(-1))

    c3 = p['conv3']
    h2, xl, xr = _tc_mid(acc2, row(c2['bias']), row(p['g2']), row(p['b2']),
                         h1, c3['Wl'], row(c3['bl']), c3['Wr'], row(c3['br']), OUT)
    acc3 = _sc_gat_kernel(1, OUT)(xl, xr, src_e, dst_e, c3['att'].reshape(-1))

    y = _tc_post(acc3, row(c3['bias']), p['Wc1'], row(p['bc1']),
                 p['Wc2'], row(p['bc2']))
    return y[:N]


# speedup vs baseline: 42.7894x; 1.0241x over previous
import functools

import jax
import jax.numpy as jnp
from jax import lax
from jax.experimental import pallas as pl
from jax.experimental.pallas import tpu as pltpu
from jax.experimental.pallas import tpu_sc as plsc

N = 10000
E = 320000
D = 128
HEADS, HC, OUT, CLS_HID = 8, 16, 64, 16

NP = 10112
EP = 330240
NC, NS = 2, 16
NW = NC * NS
EW = EP // NW
RPS = NP // NS

_BN_SCALE = 1.0 / (1.0 + 1e-5) ** 0.5

f32 = jnp.float32
i32 = jnp.int32


@functools.cache
def _sc_gat_kernel(heads, hc, G, UNROLL):
    CHUNKS = EW // G
    hwc = heads * hc
    aw = hwc + 16
    nv = hwc // 16
    vph = hc // 16
    mesh = plsc.VectorSubcoreMesh(core_axis_name="c", subcore_axis_name="s",
                                  num_cores=NC, num_subcores=NS)

    @functools.partial(
        pl.kernel,
        out_type=jax.ShapeDtypeStruct((NC, NP, aw), f32),
        mesh=mesh,
        scratch_types=[
            pltpu.VMEM((G,), i32),
            pltpu.VMEM((G,), i32),
            pltpu.VMEM((G, hwc), f32),
            pltpu.VMEM((G, hwc), f32),
            pltpu.VMEM((G, aw), f32),
            pltpu.VMEM((hwc,), f32),
            pltpu.VMEM_SHARED((NP, aw), f32),
            pltpu.SemaphoreType.DMA,
            pltpu.SemaphoreType.DMA,
        ],
        compiler_params=pltpu.CompilerParams(needs_layout_passes=False,
                                             use_tc_tiling_on_sc=False),
    )
    def kern(xl_hbm, xr_hbm, src_hbm, dst_hbm, att_hbm, out_hbm,
             idxs, idxd, bufl, bufr, val, att_v, acc, sem1, sem2):
        c = lax.axis_index("c")
        s = lax.axis_index("s")
        wid = c * NS + s
        lanes = lax.iota(i32, 16)
        zero16 = jnp.zeros((16,), f32)

        pltpu.sync_copy(att_hbm, att_v)
        att_regs = [att_v[pl.ds(16 * j, 16)] for j in range(nv)]

        def zrow(e, _):
            for j in range(aw // 16):
                val[e, pl.ds(16 * j, 16)] = zero16
            return 0

        lax.fori_loop(i32(0), i32(G), zrow, 0)
        row0 = s * i32(RPS)
        off = 0
        for sz in (G,) * (RPS // G) + ((RPS % G,) if RPS % G else ()):
            pltpu.sync_copy(val.at[pl.ds(0, sz)], acc.at[pl.ds(row0 + off, sz)])
            off += sz
        plsc.subcore_barrier()

        lane15 = jnp.full((16, 1), 15, i32)
        _gdn = lax.GatherDimensionNumbers(
            offset_dims=(), collapsed_slice_dims=(0,), start_index_map=(0,))

        def _bcast(v, idx_vec):
            return lax.gather(v, idx_vec, _gdn, (1,),
                              mode=lax.GatherScatterMode.PROMISE_IN_BOUNDS)

        def edge_body(e):
            zls = []
            ts = []
            for h in range(heads):
                acc_t = None
                for v in range(vph):
                    j = h * vph + v
                    zl = bufl[e, pl.ds(16 * j, 16)]
                    zls.append(zl)
                    z = zl + bufr[e, pl.ds(16 * j, 16)]
                    z = jnp.maximum(z, 0.2 * z)
                    t = z * att_regs[j]
                    acc_t = t if acc_t is None else acc_t + t
                ts.append(acc_t)
            tots = [_bcast(plsc.cumsum(t), lane15) for t in ts]
            alpha = jnp.full((16,), -100.0, f32)
            for h in range(heads):
                alpha = jnp.where(lanes == h, tots[h], alpha)
            wv = jnp.exp(jnp.minimum(alpha, 60.0))
            for h in range(heads):
                wb = _bcast(wv, jnp.full((16, 1), h, i32))
                for v in range(vph):
                    j = h * vph + v
                    val[e, pl.ds(16 * j, 16)] = zls[j] * wb
            val[e, pl.ds(hwc, 16)] = wv

        base = wid * i32(EW)

        def chunk(k, _):
            o = base + k * i32(G)
            pltpu.sync_copy(src_hbm.at[pl.ds(o, G)], idxs)
            pltpu.sync_copy(dst_hbm.at[pl.ds(o, G)], idxd)
            cpl = pltpu.async_copy(xl_hbm.at[idxs], bufl, sem1)
            cpr = pltpu.async_copy(xr_hbm.at[idxd], bufr, sem2)
            cpl.wait()
            cpr.wait()
            def edge_group(i, car):
                for u in range(UNROLL):
                    edge_body(UNROLL * i + u)
                return car

            lax.fori_loop(i32(0), i32(G // UNROLL), edge_group, 0)
            pltpu.sync_copy(val, acc.at[idxd], add=True)
            return 0

        lax.fori_loop(i32(0), i32(CHUNKS), chunk, 0)
        plsc.subcore_barrier()

        off = 0
        for sz in (G,) * (RPS // G) + ((RPS % G,) if RPS % G else ()):
            pltpu.sync_copy(acc.at[pl.ds(row0 + off, sz)],
                            out_hbm.at[c, pl.ds(row0 + off, sz)])
            off += sz

    return kern


def _tc_call(body, out_shapes, *args):
    return pl.pallas_call(body, out_shape=out_shapes)(*args)


def _tc_pre(x, g0, b0, wl, bl, wr, br):
    def body(x_ref, g_ref, b_ref, wl_ref, bl_ref, wr_ref, br_ref,
             h_ref, xl_ref, xr_ref):
        h = x_ref[...] * (g_ref[...] * _BN_SCALE) + b_ref[...]
        h_ref[...] = h
        xl_ref[...] = jnp.dot(h, wl_ref[...], preferred_element_type=f32) + bl_ref[...]
        xr_ref[...] = jnp.dot(h, wr_ref[...], preferred_element_type=f32) + br_ref[...]

    outs = [jax.ShapeDtypeStruct((NP, D), f32)] * 3
    return _tc_call(body, outs, x, g0, b0, wl, bl, wr, br)


def _div_den(a, heads, hc):
    hwc = heads * hc
    num = a[:, :hwc]
    den = a[:, hwc:hwc + heads]
    hh = lax.broadcasted_iota(i32, (heads, hwc), 0)
    cc = lax.broadcasted_iota(i32, (heads, hwc), 1)
    rep = jnp.where(cc // hc == hh, f32(1.0), f32(0.0))
    den_rep = jnp.dot(den, rep, preferred_element_type=f32)
    return num / (den_rep + 1e-16)


def _tc_mid(acc, bias, g, b, hprev, wl, bl, wr, br, dout):
    hwc = HEADS * HC

    def body(a_ref, bias_ref, g_ref, b_ref, hp_ref, wl_ref, bl_ref,
             wr_ref, br_ref, h_ref, xl_ref, xr_ref):
        a = a_ref[0] + a_ref[1]
        o = _div_den(a, HEADS, HC) + bias_ref[...]
        o = o * (g_ref[...] * _BN_SCALE) + b_ref[...]
        o = jnp.where(o > 0, o, jnp.exp(o) - 1.0)
        h = o + hp_ref[...]
        h_ref[...] = h
        xl_ref[...] = jnp.dot(h, wl_ref[...], preferred_element_type=f32) + bl_ref[...]
        xr_ref[...] = jnp.dot(h, wr_ref[...], preferred_element_type=f32) + br_ref[...]

    outs = [jax.ShapeDtypeStruct((NP, hwc), f32),
            jax.ShapeDtypeStruct((NP, dout), f32),
            jax.ShapeDtypeStruct((NP, dout), f32)]
    return _tc_call(body, outs, acc, bias, g, b, hprev, wl, bl, wr, br)


def _tc_post(acc, bias, wc1, bc1, wc2, bc2):
    def body(a_ref, bias_ref, w1_ref, b1_ref, w2_ref, b2_ref, y_ref):
        a = a_ref[0] + a_ref[1]
        o = _div_den(a, 1, OUT) + bias_ref[...]
        y = jnp.dot(o, w1_ref[...], preferred_element_type=f32) + b1_ref[...]
        y = jnp.maximum(y, 0.0)
        y_ref[...] = jnp.dot(y, w2_ref[...], preferred_element_type=f32) + b2_ref[...]

    outs = jax.ShapeDtypeStruct((NP, OUT), f32)
    return _tc_call(body, outs, acc, bias, wc1, bc1, wc2, bc2)


def kernel(x, edge_index, params):
    p = params
    src = edge_index[0].astype(i32)
    dst = edge_index[1].astype(i32)
    dstm = jnp.where(src == dst, N, dst)
    loops = jnp.arange(N, dtype=i32)
    padi = jnp.full((EP - E - N,), N, i32)
    src_e = jnp.concatenate([src, loops, padi])
    dst_e = jnp.concatenate([dstm, loops, padi])

    xp = jnp.pad(x.astype(f32), ((0, NP - N), (0, 0)))

    def row(v):
        return v.reshape(1, -1).astype(f32)

    c1 = p['conv1']
    h0, xl, xr = _tc_pre(xp, row(p['g0']), row(p['b0']),
                         c1['Wl'], row(c1['bl']), c1['Wr'], row(c1['br']))
    acc1 = _sc_gat_kernel(HEADS, HC, 80, 2)(xl, xr, src_e, dst_e,
                                            c1['att'].reshape(-1))

    c2 = p['conv2']
    h1, xl, xr = _tc_mid(acc1, row(c1['bias']), row(p['g1']), row(p['b1']),
                         h0, c2['Wl'], row(c2['bl']), c2['Wr'], row(c2['br']), D)
    acc2 = _sc_gat_kernel(HEADS, HC, 80, 2)(xl, xr, src_e, dst_e,
                                            c2['att'].reshape(-1))

    c3 = p['conv3']
    h2, xl, xr = _tc_mid(acc2, row(c2['bias']), row(p['g2']), row(p['b2']),
                         h1, c3['Wl'], row(c3['bl']), c3['Wr'], row(c3['br']), OUT)
    acc3 = _sc_gat_kernel(1, OUT, 120, 4)(xl, xr, src_e, dst_e,
                                          c3['att'].reshape(-1))

    y = _tc_post(acc3, row(c3['bias']), p['Wc1'], row(p['bc1']),
                 p['Wc2'], row(p['bc2']))
    return y[:N]


# speedup vs baseline: 52.1950x; 1.2198x over previous
import functools

import jax
import jax.numpy as jnp
from jax import lax
from jax.experimental import pallas as pl
from jax.experimental.pallas import tpu as pltpu
from jax.experimental.pallas import tpu_sc as plsc

N = 10000
E = 320000
D = 128
HEADS, HC, OUT, CLS_HID = 8, 16, 64, 16

NP = 10112
EP = 330240
NC, NS = 2, 16
NW = NC * NS
EW = EP // NW
RPS = NP // NS

_BN_SCALE = 1.0 / (1.0 + 1e-5) ** 0.5

f32 = jnp.float32
i32 = jnp.int32


@functools.cache
def _sc_gat_kernel(heads, hc, G, UNROLL):
    CHUNKS = EW // G
    hwc = heads * hc
    aw = hwc + 16
    nv = hwc // 16
    vph = hc // 16
    mesh = plsc.VectorSubcoreMesh(core_axis_name="c", subcore_axis_name="s",
                                  num_cores=NC, num_subcores=NS)

    @functools.partial(
        pl.kernel,
        out_type=jax.ShapeDtypeStruct((NC, NP, aw), f32),
        mesh=mesh,
        scratch_types=[
            pltpu.VMEM((G,), i32), pltpu.VMEM((G,), i32),
            pltpu.VMEM((G,), i32), pltpu.VMEM((G,), i32),
            pltpu.VMEM((G,), i32), pltpu.VMEM((G,), i32),
            pltpu.VMEM((G, hwc), f32), pltpu.VMEM((G, hwc), f32),
            pltpu.VMEM((G, hwc), f32), pltpu.VMEM((G, hwc), f32),
            pltpu.VMEM((G, aw), f32), pltpu.VMEM((G, aw), f32),
            pltpu.VMEM((hwc,), f32),
            pltpu.VMEM_SHARED((NP, aw), f32),
            pltpu.SemaphoreType.DMA, pltpu.SemaphoreType.DMA,
            pltpu.SemaphoreType.DMA, pltpu.SemaphoreType.DMA,
            pltpu.SemaphoreType.DMA, pltpu.SemaphoreType.DMA,
        ],
        compiler_params=pltpu.CompilerParams(needs_layout_passes=False,
                                             use_tc_tiling_on_sc=False),
    )
    def kern(xl_hbm, xr_hbm, src_hbm, dst_hbm, att_hbm, out_hbm,
             idxs0, idxs1, idxd0, idxd1, idxc0, idxc1,
             bufl0, bufl1, bufr0, bufr1, val0, val1, att_v, acc,
             seml0, seml1, semr0, semr1, semc0, semc1):
        idxs = (idxs0, idxs1)
        idxd = (idxd0, idxd1)
        idxc = (idxc0, idxc1)
        bufl = (bufl0, bufl1)
        bufr = (bufr0, bufr1)
        val = (val0, val1)
        seml = (seml0, seml1)
        semr = (semr0, semr1)
        semc = (semc0, semc1)
        c = lax.axis_index("c")
        s = lax.axis_index("s")
        wid = c * NS + s
        lanes = lax.iota(i32, 16)
        zero16 = jnp.zeros((16,), f32)

        pltpu.sync_copy(att_hbm, att_v)
        att_regs = [att_v[pl.ds(16 * j, 16)] for j in range(nv)]

        def zrow(e, _):
            for j in range(aw // 16):
                val0[e, pl.ds(16 * j, 16)] = zero16
            return 0

        lax.fori_loop(i32(0), i32(G), zrow, 0)
        row0 = s * i32(RPS)
        off = 0
        for sz in (G,) * (RPS // G) + ((RPS % G,) if RPS % G else ()):
            pltpu.sync_copy(val0.at[pl.ds(0, sz)], acc.at[pl.ds(row0 + off, sz)])
            off += sz
        plsc.subcore_barrier()

        lane15 = jnp.full((16, 1), 15, i32)
        _gdn = lax.GatherDimensionNumbers(
            offset_dims=(), collapsed_slice_dims=(0,), start_index_map=(0,))

        def _bcast(v, idx_vec):
            return lax.gather(v, idx_vec, _gdn, (1,),
                              mode=lax.GatherScatterMode.PROMISE_IN_BOUNDS)

        def edge_body(bl, br, vl, e):
            zls = []
            ts = []
            for h in range(heads):
                acc_t = None
                for v in range(vph):
                    j = h * vph + v
                    zl = bl[e, pl.ds(16 * j, 16)]
                    zls.append(zl)
                    z = zl + br[e, pl.ds(16 * j, 16)]
                    z = jnp.maximum(z, 0.2 * z)
                    t = z * att_regs[j]
                    acc_t = t if acc_t is None else acc_t + t
                ts.append(acc_t)
            tots = [_bcast(plsc.cumsum(t), lane15) for t in ts]
            alpha = jnp.full((16,), -100.0, f32)
            for h in range(heads):
                alpha = jnp.where(lanes == h, tots[h], alpha)
            wv = jnp.exp(jnp.minimum(alpha, 60.0))
            for h in range(heads):
                wb = _bcast(wv, jnp.full((16, 1), h, i32))
                for v in range(vph):
                    j = h * vph + v
                    vl[e, pl.ds(16 * j, 16)] = zls[j] * wb
            vl[e, pl.ds(hwc, 16)] = wv

        base = wid * i32(EW)

        def load(k, st):
            o = base + k * i32(G)
            pltpu.sync_copy(src_hbm.at[pl.ds(o, G)], idxs[st])
            pltpu.sync_copy(dst_hbm.at[pl.ds(o, G)], idxd[st])
            pltpu.async_copy(xl_hbm.at[idxs[st]], bufl[st], seml[st])
            pltpu.async_copy(xr_hbm.at[idxd[st]], bufr[st], semr[st])

        def wait_gather(st):
            pltpu.make_async_copy(xl_hbm.at[idxs[st]], bufl[st], seml[st]).wait()
            pltpu.make_async_copy(xr_hbm.at[idxd[st]], bufr[st], semr[st]).wait()

        def wait_scatter(st):
            pltpu.make_async_copy(val[st], acc.at[idxc[st]], semc[st]).wait()

        def compute_scatter(st):
            def edge_group(i, car):
                for u in range(UNROLL):
                    edge_body(bufl[st], bufr[st], val[st], UNROLL * i + u)
                return car

            lax.fori_loop(i32(0), i32(G // UNROLL), edge_group, 0)
            for r in range(G // 16):
                idxc[st][pl.ds(16 * r, 16)] = idxd[st][pl.ds(16 * r, 16)]
            pltpu.async_copy(val[st], acc.at[idxc[st]], semc[st], add=True)

        K2 = CHUNKS // 2
        load(i32(0), 0)

        def outer(k2, car):
            ka = 2 * k2
            wait_gather(0)
            load(ka + 1, 1)
            pl.when(k2 > 0)(lambda: wait_scatter(0))
            compute_scatter(0)
            wait_gather(1)
            pl.when(ka + 2 < i32(CHUNKS))(lambda: load(ka + 2, 0))
            pl.when(k2 > 0)(lambda: wait_scatter(1))
            compute_scatter(1)
            return car

        lax.fori_loop(i32(0), i32(K2), outer, 0)
        if CHUNKS % 2:
            wait_gather(0)
            wait_scatter(0)
            compute_scatter(0)
        wait_scatter(0)
        wait_scatter(1)
        plsc.subcore_barrier()
        plsc.subcore_barrier()

        off = 0
        for sz in (G,) * (RPS // G) + ((RPS % G,) if RPS % G else ()):
            pltpu.sync_copy(acc.at[pl.ds(row0 + off, sz)],
                            out_hbm.at[c, pl.ds(row0 + off, sz)])
            off += sz

    return kern


def _tc_call(body, out_shapes, *args):
    return pl.pallas_call(body, out_shape=out_shapes)(*args)


def _tc_pre(x, g0, b0, wl, bl, wr, br):
    def body(x_ref, g_ref, b_ref, wl_ref, bl_ref, wr_ref, br_ref,
             h_ref, xl_ref, xr_ref):
        h = x_ref[...] * (g_ref[...] * _BN_SCALE) + b_ref[...]
        h_ref[...] = h
        xl_ref[...] = jnp.dot(h, wl_ref[...], preferred_element_type=f32) + bl_ref[...]
        xr_ref[...] = jnp.dot(h, wr_ref[...], preferred_element_type=f32) + br_ref[...]

    outs = [jax.ShapeDtypeStruct((NP, D), f32)] * 3
    return _tc_call(body, outs, x, g0, b0, wl, bl, wr, br)


def _div_den(a, heads, hc):
    hwc = heads * hc
    num = a[:, :hwc]
    den = a[:, hwc:hwc + heads]
    hh = lax.broadcasted_iota(i32, (heads, hwc), 0)
    cc = lax.broadcasted_iota(i32, (heads, hwc), 1)
    rep = jnp.where(cc // hc == hh, f32(1.0), f32(0.0))
    den_rep = jnp.dot(den, rep, preferred_element_type=f32)
    return num / (den_rep + 1e-16)


def _tc_mid(acc, bias, g, b, hprev, wl, bl, wr, br, dout):
    hwc = HEADS * HC

    def body(a_ref, bias_ref, g_ref, b_ref, hp_ref, wl_ref, bl_ref,
             wr_ref, br_ref, h_ref, xl_ref, xr_ref):
        a = a_ref[0] + a_ref[1]
        o = _div_den(a, HEADS, HC) + bias_ref[...]
        o = o * (g_ref[...] * _BN_SCALE) + b_ref[...]
        o = jnp.where(o > 0, o, jnp.exp(o) - 1.0)
        h = o + hp_ref[...]
        h_ref[...] = h
        xl_ref[...] = jnp.dot(h, wl_ref[...], preferred_element_type=f32) + bl_ref[...]
        xr_ref[...] = jnp.dot(h, wr_ref[...], preferred_element_type=f32) + br_ref[...]

    outs = [jax.ShapeDtypeStruct((NP, hwc), f32),
            jax.ShapeDtypeStruct((NP, dout), f32),
            jax.ShapeDtypeStruct((NP, dout), f32)]
    return _tc_call(body, outs, acc, bias, g, b, hprev, wl, bl, wr, br)


def _tc_post(acc, bias, wc1, bc1, wc2, bc2):
    def body(a_ref, bias_ref, w1_ref, b1_ref, w2_ref, b2_ref, y_ref):
        a = a_ref[0] + a_ref[1]
        o = _div_den(a, 1, OUT) + bias_ref[...]
        y = jnp.dot(o, w1_ref[...], preferred_element_type=f32) + b1_ref[...]
        y = jnp.maximum(y, 0.0)
        y_ref[...] = jnp.dot(y, w2_ref[...], preferred_element_type=f32) + b2_ref[...]

    outs = jax.ShapeDtypeStruct((NP, OUT), f32)
    return _tc_call(body, outs, acc, bias, wc1, bc1, wc2, bc2)


def kernel(x, edge_index, params):
    p = params
    src = edge_index[0].astype(i32)
    dst = edge_index[1].astype(i32)
    dstm = jnp.where(src == dst, N, dst)
    loops = jnp.arange(N, dtype=i32)
    padi = jnp.full((EP - E - N,), N, i32)
    src_e = jnp.concatenate([src, loops, padi])
    dst_e = jnp.concatenate([dstm, loops, padi])

    xp = jnp.pad(x.astype(f32), ((0, NP - N), (0, 0)))

    def row(v):
        return v.reshape(1, -1).astype(f32)

    c1 = p['conv1']
    h0, xl, xr = _tc_pre(xp, row(p['g0']), row(p['b0']),
                         c1['Wl'], row(c1['bl']), c1['Wr'], row(c1['br']))
    acc1 = _sc_gat_kernel(HEADS, HC, 48, 2)(xl, xr, src_e, dst_e,
                                            c1['att'].reshape(-1))

    c2 = p['conv2']
    h1, xl, xr = _tc_mid(acc1, row(c1['bias']), row(p['g1']), row(p['b1']),
                         h0, c2['Wl'], row(c2['bl']), c2['Wr'], row(c2['br']), D)
    acc2 = _sc_gat_kernel(HEADS, HC, 48, 2)(xl, xr, src_e, dst_e,
                                            c2['att'].reshape(-1))

    c3 = p['conv3']
    h2, xl, xr = _tc_mid(acc2, row(c2['bias']), row(p['g2']), row(p['b2']),
                         h1, c3['Wl'], row(c3['bl']), c3['Wr'], row(c3['br']), OUT)
    acc3 = _sc_gat_kernel(1, OUT, 80, 4)(xl, xr, src_e, dst_e,
                                          c3['att'].reshape(-1))

    y = _tc_post(acc3, row(c3['bias']), p['Wc1'], row(p['bc1']),
                 p['Wc2'], row(p['bc2']))
    return y[:N]


# speedup vs baseline: 52.2612x; 1.0013x over previous
import functools

import jax
import jax.numpy as jnp
from jax import lax
from jax.experimental import pallas as pl
from jax.experimental.pallas import tpu as pltpu
from jax.experimental.pallas import tpu_sc as plsc

N = 10000
E = 320000
D = 128
HEADS, HC, OUT, CLS_HID = 8, 16, 64, 16

NP = 10112
EP = 330240
NC, NS = 2, 16
NW = NC * NS
EW = EP // NW
RPS = NP // NS

_BN_SCALE = 1.0 / (1.0 + 1e-5) ** 0.5

f32 = jnp.float32
i32 = jnp.int32


@functools.cache
def _sc_gat_kernel(heads, hc, G, UNROLL):
    CHUNKS = EW // G
    hwc = heads * hc
    aw = hwc + 16
    nv = hwc // 16
    vph = hc // 16
    mesh = plsc.VectorSubcoreMesh(core_axis_name="c", subcore_axis_name="s",
                                  num_cores=NC, num_subcores=NS)

    @functools.partial(
        pl.kernel,
        out_type=jax.ShapeDtypeStruct((NC, NP, aw), f32),
        mesh=mesh,
        scratch_types=[
            pltpu.VMEM((G,), i32), pltpu.VMEM((G,), i32),
            pltpu.VMEM((G,), i32), pltpu.VMEM((G,), i32),
            pltpu.VMEM((G,), i32), pltpu.VMEM((G,), i32),
            pltpu.VMEM((G, hwc), f32), pltpu.VMEM((G, hwc), f32),
            pltpu.VMEM((G, hwc), f32), pltpu.VMEM((G, hwc), f32),
            pltpu.VMEM((G, aw), f32), pltpu.VMEM((G, aw), f32),
            pltpu.VMEM((hwc,), f32),
            pltpu.VMEM_SHARED((NP, aw), f32),
            pltpu.SemaphoreType.DMA, pltpu.SemaphoreType.DMA,
            pltpu.SemaphoreType.DMA, pltpu.SemaphoreType.DMA,
            pltpu.SemaphoreType.DMA, pltpu.SemaphoreType.DMA,
        ],
        compiler_params=pltpu.CompilerParams(needs_layout_passes=False,
                                             use_tc_tiling_on_sc=False),
    )
    def kern(xl_hbm, xr_hbm, src_hbm, dst_hbm, att_hbm, out_hbm,
             idxs0, idxs1, idxd0, idxd1, idxc0, idxc1,
             bufl0, bufl1, bufr0, bufr1, val0, val1, att_v, acc,
             seml0, seml1, semr0, semr1, semc0, semc1):
        idxs = (idxs0, idxs1)
        idxd = (idxd0, idxd1)
        idxc = (idxc0, idxc1)
        bufl = (bufl0, bufl1)
        bufr = (bufr0, bufr1)
        val = (val0, val1)
        seml = (seml0, seml1)
        semr = (semr0, semr1)
        semc = (semc0, semc1)
        c = lax.axis_index("c")
        s = lax.axis_index("s")
        wid = c * NS + s
        lanes = lax.iota(i32, 16)
        zero16 = jnp.zeros((16,), f32)

        pltpu.sync_copy(att_hbm, att_v)
        att_regs = [att_v[pl.ds(16 * j, 16)] for j in range(nv)]

        def zrow(e, _):
            for j in range(aw // 16):
                val0[e, pl.ds(16 * j, 16)] = zero16
            return 0

        lax.fori_loop(i32(0), i32(G), zrow, 0)
        row0 = s * i32(RPS)
        off = 0
        for sz in (G,) * (RPS // G) + ((RPS % G,) if RPS % G else ()):
            pltpu.sync_copy(val0.at[pl.ds(0, sz)], acc.at[pl.ds(row0 + off, sz)])
            off += sz
        plsc.subcore_barrier()

        lane15 = jnp.full((16, 1), 15, i32)
        _gdn = lax.GatherDimensionNumbers(
            offset_dims=(), collapsed_slice_dims=(0,), start_index_map=(0,))

        def _bcast(v, idx_vec):
            return lax.gather(v, idx_vec, _gdn, (1,),
                              mode=lax.GatherScatterMode.PROMISE_IN_BOUNDS)

        def edge_body(bl, br, vl, e):
            zls = []
            ts = []
            for h in range(heads):
                acc_t = None
                for v in range(vph):
                    j = h * vph + v
                    zl = bl[e, pl.ds(16 * j, 16)]
                    zls.append(zl)
                    z = zl + br[e, pl.ds(16 * j, 16)]
                    z = jnp.maximum(z, 0.2 * z)
                    t = z * att_regs[j]
                    acc_t = t if acc_t is None else acc_t + t
                ts.append(acc_t)
            tots = [_bcast(plsc.cumsum(t), lane15) for t in ts]
            alpha = jnp.full((16,), -100.0, f32)
            for h in range(heads):
                alpha = jnp.where(lanes == h, tots[h], alpha)
            wv = jnp.exp(jnp.minimum(alpha, 60.0))
            for h in range(heads):
                wb = _bcast(wv, jnp.full((16, 1), h, i32))
                for v in range(vph):
                    j = h * vph + v
                    vl[e, pl.ds(16 * j, 16)] = zls[j] * wb
            vl[e, pl.ds(hwc, 16)] = wv

        base = wid * i32(EW)

        def load(k, st):
            o = base + k * i32(G)
            pltpu.sync_copy(src_hbm.at[pl.ds(o, G)], idxs[st])
            pltpu.sync_copy(dst_hbm.at[pl.ds(o, G)], idxd[st])
            pltpu.async_copy(xl_hbm.at[idxs[st]], bufl[st], seml[st])
            pltpu.async_copy(xr_hbm.at[idxd[st]], bufr[st], semr[st])

        def wait_gather(st):
            pltpu.make_async_copy(xl_hbm.at[idxs[st]], bufl[st], seml[st]).wait()
            pltpu.make_async_copy(xr_hbm.at[idxd[st]], bufr[st], semr[st]).wait()

        def wait_scatter(st):
            pltpu.make_async_copy(val[st], acc.at[idxc[st]], semc[st]).wait()

        def compute_scatter(st):
            def edge_group(i, car):
                for u in range(UNROLL):
                    edge_body(bufl[st], bufr[st], val[st], UNROLL * i + u)
                return car

            lax.fori_loop(i32(0), i32(G // UNROLL), edge_group, 0)
            for r in range(G // 16):
                idxc[st][pl.ds(16 * r, 16)] = idxd[st][pl.ds(16 * r, 16)]
            pltpu.async_copy(val[st], acc.at[idxc[st]], semc[st], add=True)

        K2 = CHUNKS // 2
        load(i32(0), 0)

        def outer(k2, car):
            ka = 2 * k2
            wait_gather(0)
            load(ka + 1, 1)
            pl.when(k2 > 0)(lambda: wait_scatter(0))
            compute_scatter(0)
            wait_gather(1)
            pl.when(ka + 2 < i32(CHUNKS))(lambda: load(ka + 2, 0))
            pl.when(k2 > 0)(lambda: wait_scatter(1))
            compute_scatter(1)
            return car

        lax.fori_loop(i32(0), i32(K2), outer, 0)
        if CHUNKS % 2:
            wait_gather(0)
            wait_scatter(0)
            compute_scatter(0)
        wait_scatter(0)
        wait_scatter(1)
        plsc.subcore_barrier()
        plsc.subcore_barrier()

        off = 0
        for sz in (G,) * (RPS // G) + ((RPS % G,) if RPS % G else ()):
            pltpu.sync_copy(acc.at[pl.ds(row0 + off, sz)],
                            out_hbm.at[c, pl.ds(row0 + off, sz)])
            off += sz

    return kern


def _tc_call(body, out_shapes, *args):
    return pl.pallas_call(body, out_shape=out_shapes)(*args)


def _tc_pre(x, g0, b0, wl, bl, wr, br):
    def body(x_ref, g_ref, b_ref, wl_ref, bl_ref, wr_ref, br_ref,
             h_ref, xl_ref, xr_ref):
        h = x_ref[...] * (g_ref[...] * _BN_SCALE) + b_ref[...]
        h_ref[...] = h
        xl_ref[...] = jnp.dot(h, wl_ref[...], preferred_element_type=f32) + bl_ref[...]
        xr_ref[...] = jnp.dot(h, wr_ref[...], preferred_element_type=f32) + br_ref[...]

    outs = [jax.ShapeDtypeStruct((NP, D), f32)] * 3
    return _tc_call(body, outs, x, g0, b0, wl, bl, wr, br)


def _div_den(a, heads, hc):
    hwc = heads * hc
    num = a[:, :hwc]
    den = a[:, hwc:hwc + heads]
    hh = lax.broadcasted_iota(i32, (heads, hwc), 0)
    cc = lax.broadcasted_iota(i32, (heads, hwc), 1)
    rep = jnp.where(cc // hc == hh, f32(1.0), f32(0.0))
    den_rep = jnp.dot(den, rep, preferred_element_type=f32)
    return num / (den_rep + 1e-16)


def _tc_mid(acc, bias, g, b, hprev, wl, bl, wr, br, dout):
    hwc = HEADS * HC

    def body(a_ref, bias_ref, g_ref, b_ref, hp_ref, wl_ref, bl_ref,
             wr_ref, br_ref, h_ref, xl_ref, xr_ref):
        a = a_ref[0] + a_ref[1]
        o = _div_den(a, HEADS, HC) + bias_ref[...]
        o = o * (g_ref[...] * _BN_SCALE) + b_ref[...]
        o = jnp.where(o > 0, o, jnp.exp(o) - 1.0)
        h = o + hp_ref[...]
        h_ref[...] = h
        xl_ref[...] = jnp.dot(h, wl_ref[...], preferred_element_type=f32) + bl_ref[...]
        xr_ref[...] = jnp.dot(h, wr_ref[...], preferred_element_type=f32) + br_ref[...]

    outs = [jax.ShapeDtypeStruct((NP, hwc), f32),
            jax.ShapeDtypeStruct((NP, dout), f32),
            jax.ShapeDtypeStruct((NP, dout), f32)]
    return _tc_call(body, outs, acc, bias, g, b, hprev, wl, bl, wr, br)


def _tc_post(acc, bias, wc1, bc1, wc2, bc2):
    def body(a_ref, bias_ref, w1_ref, b1_ref, w2_ref, b2_ref, y_ref):
        a = a_ref[0] + a_ref[1]
        o = _div_den(a, 1, OUT) + bias_ref[...]
        y = jnp.dot(o, w1_ref[...], preferred_element_type=f32) + b1_ref[...]
        y = jnp.maximum(y, 0.0)
        y_ref[...] = jnp.dot(y, w2_ref[...], preferred_element_type=f32) + b2_ref[...]

    outs = jax.ShapeDtypeStruct((NP, OUT), f32)
    return _tc_call(body, outs, acc, bias, wc1, bc1, wc2, bc2)


def kernel(x, edge_index, params):
    p = params
    src = edge_index[0].astype(i32)
    dst = edge_index[1].astype(i32)
    dstm = jnp.where(src == dst, N, dst)
    loops = jnp.arange(N, dtype=i32)
    padi = jnp.full((EP - E - N,), N, i32)
    src_e = jnp.concatenate([src, loops, padi])
    dst_e = jnp.concatenate([dstm, loops, padi])

    xp = jnp.pad(x.astype(f32), ((0, NP - N), (0, 0)))

    def row(v):
        return v.reshape(1, -1).astype(f32)

    c1 = p['conv1']
    h0, xl, xr = _tc_pre(xp, row(p['g0']), row(p['b0']),
                         c1['Wl'], row(c1['bl']), c1['Wr'], row(c1['br']))
    acc1 = _sc_gat_kernel(HEADS, HC, 48, 4)(xl, xr, src_e, dst_e,
                                            c1['att'].reshape(-1))

    c2 = p['conv2']
    h1, xl, xr = _tc_mid(acc1, row(c1['bias']), row(p['g1']), row(p['b1']),
                         h0, c2['Wl'], row(c2['bl']), c2['Wr'], row(c2['br']), D)
    acc2 = _sc_gat_kernel(HEADS, HC, 48, 4)(xl, xr, src_e, dst_e,
                                            c2['att'].reshape(-1))

    c3 = p['conv3']
    h2, xl, xr = _tc_mid(acc2, row(c2['bias']), row(p['g2']), row(p['b2']),
                         h1, c3['Wl'], row(c3['bl']), c3['Wr'], row(c3['br']), OUT)
    acc3 = _sc_gat_kernel(1, OUT, 80, 8)(xl, xr, src_e, dst_e,
                                          c3['att'].reshape(-1))

    y = _tc_post(acc3, row(c3['bias']), p['Wc1'], row(p['bc1']),
                 p['Wc2'], row(p['bc2']))
    return y[:N]


# speedup vs baseline: 69.1014x; 1.3222x over previous
import functools

import jax
import jax.numpy as jnp
from jax import lax
from jax.experimental import pallas as pl
from jax.experimental.pallas import tpu as pltpu
from jax.experimental.pallas import tpu_sc as plsc

N = 10000
E = 320000
D = 128
HEADS, HC, OUT, CLS_HID = 8, 16, 64, 16

NP = 10112
EP = 330240
NC, NS = 2, 16
NW = NC * NS
EW = EP // NW
RPS = NP // NS

_BN_SCALE = 1.0 / (1.0 + 1e-5) ** 0.5

f32 = jnp.float32
i32 = jnp.int32


@functools.cache
def _sc_gat_kernel(heads, hc, G, UNROLL):
    CHUNKS = EW // G
    hwc = heads * hc
    aw = hwc + 16
    nv = hwc // 16
    vph = hc // 16
    mesh = plsc.VectorSubcoreMesh(core_axis_name="c", subcore_axis_name="s",
                                  num_cores=NC, num_subcores=NS)

    @functools.partial(
        pl.kernel,
        out_type=jax.ShapeDtypeStruct((NC, NP, aw), f32),
        mesh=mesh,
        scratch_types=[
            pltpu.VMEM((G,), i32), pltpu.VMEM((G,), i32),
            pltpu.VMEM((G,), i32), pltpu.VMEM((G,), i32),
            pltpu.VMEM((G,), i32), pltpu.VMEM((G,), i32),
            pltpu.VMEM((G, hwc), f32), pltpu.VMEM((G, hwc), f32),
            pltpu.VMEM((G, hwc), f32), pltpu.VMEM((G, hwc), f32),
            pltpu.VMEM((G, aw), f32), pltpu.VMEM((G, aw), f32),
            pltpu.VMEM((hwc,), f32),
            pltpu.VMEM_SHARED((NP, aw), f32),
            pltpu.SemaphoreType.DMA, pltpu.SemaphoreType.DMA,
            pltpu.SemaphoreType.DMA, pltpu.SemaphoreType.DMA,
            pltpu.SemaphoreType.DMA, pltpu.SemaphoreType.DMA,
            pltpu.SemaphoreType.DMA, pltpu.SemaphoreType.DMA,
        ],
        compiler_params=pltpu.CompilerParams(needs_layout_passes=False,
                                             use_tc_tiling_on_sc=False),
    )
    def kern(xl_hbm, xr_hbm, src_hbm, dst_hbm, att_hbm, out_hbm,
             idxs0, idxs1, idxd0, idxd1, idxc0, idxc1,
             bufl0, bufl1, bufr0, bufr1, val0, val1, att_v, acc,
             seml0, seml1, semr0, semr1, semc0, semc1, semi0, semi1):
        idxs = (idxs0, idxs1)
        idxd = (idxd0, idxd1)
        idxc = (idxc0, idxc1)
        bufl = (bufl0, bufl1)
        bufr = (bufr0, bufr1)
        val = (val0, val1)
        seml = (seml0, seml1)
        semr = (semr0, semr1)
        semc = (semc0, semc1)
        semi = (semi0, semi1)
        c = lax.axis_index("c")
        s = lax.axis_index("s")
        wid = c * NS + s
        lanes = lax.iota(i32, 16)
        zero16 = jnp.zeros((16,), f32)

        pltpu.sync_copy(att_hbm, att_v)
        att_regs = [att_v[pl.ds(16 * j, 16)] for j in range(nv)]

        def zrow(e, _):
            for j in range(aw // 16):
                val0[e, pl.ds(16 * j, 16)] = zero16
            return 0

        lax.fori_loop(i32(0), i32(G), zrow, 0)
        row0 = s * i32(RPS)
        off = 0
        for sz in (G,) * (RPS // G) + ((RPS % G,) if RPS % G else ()):
            pltpu.sync_copy(val0.at[pl.ds(0, sz)], acc.at[pl.ds(row0 + off, sz)])
            off += sz
        plsc.subcore_barrier()

        lane15 = jnp.full((16, 1), 15, i32)
        _gdn = lax.GatherDimensionNumbers(
            offset_dims=(), collapsed_slice_dims=(0,), start_index_map=(0,))

        def _bcast(v, idx_vec):
            return lax.gather(v, idx_vec, _gdn, (1,),
                              mode=lax.GatherScatterMode.PROMISE_IN_BOUNDS)

        def edge_body(bl, br, vl, e):
            zls = []
            ts = []
            for h in range(heads):
                acc_t = None
                for v in range(vph):
                    j = h * vph + v
                    zl = bl[e, pl.ds(16 * j, 16)]
                    zls.append(zl)
                    z = zl + br[e, pl.ds(16 * j, 16)]
                    z = jnp.maximum(z, 0.2 * z)
                    t = z * att_regs[j]
                    acc_t = t if acc_t is None else acc_t + t
                ts.append(acc_t)
            tots = [_bcast(plsc.cumsum(t), lane15) for t in ts]
            alpha = jnp.full((16,), -100.0, f32)
            for h in range(heads):
                alpha = jnp.where(lanes == h, tots[h], alpha)
            wv = jnp.exp(jnp.minimum(alpha, 60.0))
            for h in range(heads):
                wb = _bcast(wv, jnp.full((16, 1), h, i32))
                for v in range(vph):
                    j = h * vph + v
                    vl[e, pl.ds(16 * j, 16)] = zls[j] * wb
            vl[e, pl.ds(hwc, 16)] = wv

        base = wid * i32(EW)
        assert CHUNKS % 2 == 1 and CHUNKS >= 5

        def idx_copy(k, st):
            o = base + k * i32(G)
            pltpu.async_copy(src_hbm.at[pl.ds(o, G)], idxs[st], semi[st])
            pltpu.async_copy(dst_hbm.at[pl.ds(o, G)], idxd[st], semi[st])

        def wait_idx(st):
            pltpu.make_async_copy(src_hbm.at[pl.ds(base, G)], idxs[st],
                                  semi[st]).wait()
            pltpu.make_async_copy(dst_hbm.at[pl.ds(base, G)], idxd[st],
                                  semi[st]).wait()

        def gather(st):
            pltpu.async_copy(xl_hbm.at[idxs[st]], bufl[st], seml[st])
            pltpu.async_copy(xr_hbm.at[idxd[st]], bufr[st], semr[st])

        def wait_gather(st):
            pltpu.make_async_copy(xl_hbm.at[idxs[st]], bufl[st], seml[st]).wait()
            pltpu.make_async_copy(xr_hbm.at[idxd[st]], bufr[st], semr[st]).wait()

        def wait_scatter(st):
            pltpu.make_async_copy(val[st], acc.at[idxc[st]], semc[st]).wait()

        def body(k, k2, st):
            wait_gather(st)
            pl.when(k2 > 0)(lambda: wait_scatter(st))
            for r in range(G // 16):
                idxc[st][pl.ds(16 * r, 16)] = idxd[st][pl.ds(16 * r, 16)]
            pl.when(k + 2 < i32(CHUNKS))(lambda: idx_copy(k + 2, st))

            def launch_next():
                wait_idx(1 - st)
                gather(1 - st)

            pl.when(k + 1 < i32(CHUNKS))(launch_next)

            def edge_group(i, car):
                for u in range(UNROLL):
                    edge_body(bufl[st], bufr[st], val[st], UNROLL * i + u)
                return car

            lax.fori_loop(i32(0), i32(G // UNROLL), edge_group, 0)
            pltpu.async_copy(val[st], acc.at[idxc[st]], semc[st], add=True)

        K2 = CHUNKS // 2
        idx_copy(i32(0), 0)
        idx_copy(i32(1), 1)
        wait_idx(0)
        gather(0)

        def outer(k2, car):
            body(2 * k2, k2, 0)
            body(2 * k2 + 1, k2, 1)
            return car

        lax.fori_loop(i32(0), i32(K2), outer, 0)
        wait_gather(0)
        wait_scatter(0)
        for r in range(G // 16):
            idxc[0][pl.ds(16 * r, 16)] = idxd[0][pl.ds(16 * r, 16)]

        def edge_group_tail(i, car):
            for u in range(UNROLL):
                edge_body(bufl[0], bufr[0], val[0], UNROLL * i + u)
            return car

        lax.fori_loop(i32(0), i32(G // UNROLL), edge_group_tail, 0)
        pltpu.async_copy(val[0], acc.at[idxc[0]], semc[0], add=True)
        wait_scatter(1)
        wait_scatter(0)
        plsc.subcore_barrier()
        plsc.subcore_barrier()

        off = 0
        for sz in (G,) * (RPS // G) + ((RPS % G,) if RPS % G else ()):
            pltpu.sync_copy(acc.at[pl.ds(row0 + off, sz)],
                            out_hbm.at[c, pl.ds(row0 + off, sz)])
            off += sz

    return kern


def _tc_call(body, out_shapes, *args):
    return pl.pallas_call(body, out_shape=out_shapes)(*args)


def _tc_pre(x, g0, b0, wl, bl, wr, br):
    def body(x_ref, g_ref, b_ref, wl_ref, bl_ref, wr_ref, br_ref,
             h_ref, xl_ref, xr_ref):
        h = x_ref[...] * (g_ref[...] * _BN_SCALE) + b_ref[...]
        h_ref[...] = h
        xl_ref[...] = jnp.dot(h, wl_ref[...], preferred_element_type=f32) + bl_ref[...]
        xr_ref[...] = jnp.dot(h, wr_ref[...], preferred_element_type=f32) + br_ref[...]

    outs = [jax.ShapeDtypeStruct((NP, D), f32)] * 3
    return _tc_call(body, outs, x, g0, b0, wl, bl, wr, br)


def _div_den(a, heads, hc):
    hwc = heads * hc
    num = a[:, :hwc]
    den = a[:, hwc:hwc + heads]
    hh = lax.broadcasted_iota(i32, (heads, hwc), 0)
    cc = lax.broadcasted_iota(i32, (heads, hwc), 1)
    rep = jnp.where(cc // hc == hh, f32(1.0), f32(0.0))
    den_rep = jnp.dot(den, rep, preferred_element_type=f32)
    return num / (den_rep + 1e-16)


def _tc_mid(acc, bias, g, b, hprev, wl, bl, wr, br, dout):
    hwc = HEADS * HC

    def body(a_ref, bias_ref, g_ref, b_ref, hp_ref, wl_ref, bl_ref,
             wr_ref, br_ref, h_ref, xl_ref, xr_ref):
        a = a_ref[0] + a_ref[1]
        o = _div_den(a, HEADS, HC) + bias_ref[...]
        o = o * (g_ref[...] * _BN_SCALE) + b_ref[...]
        o = jnp.where(o > 0, o, jnp.exp(o) - 1.0)
        h = o + hp_ref[...]
        h_ref[...] = h
        xl_ref[...] = jnp.dot(h, wl_ref[...], preferred_element_type=f32) + bl_ref[...]
        xr_ref[...] = jnp.dot(h, wr_ref[...], preferred_element_type=f32) + br_ref[...]

    outs = [jax.ShapeDtypeStruct((NP, hwc), f32),
            jax.ShapeDtypeStruct((NP, dout), f32),
            jax.ShapeDtypeStruct((NP, dout), f32)]
    return _tc_call(body, outs, acc, bias, g, b, hprev, wl, bl, wr, br)


def _tc_post(acc, bias, wc1, bc1, wc2, bc2):
    def body(a_ref, bias_ref, w1_ref, b1_ref, w2_ref, b2_ref, y_ref):
        a = a_ref[0] + a_ref[1]
        o = _div_den(a, 1, OUT) + bias_ref[...]
        y = jnp.dot(o, w1_ref[...], preferred_element_type=f32) + b1_ref[...]
        y = jnp.maximum(y, 0.0)
        y_ref[...] = jnp.dot(y, w2_ref[...], preferred_element_type=f32) + b2_ref[...]

    outs = jax.ShapeDtypeStruct((NP, OUT), f32)
    return _tc_call(body, outs, acc, bias, wc1, bc1, wc2, bc2)


def kernel(x, edge_index, params):
    p = params
    src = edge_index[0].astype(i32)
    dst = edge_index[1].astype(i32)
    dstm = jnp.where(src == dst, N, dst)
    loops = jnp.arange(N, dtype=i32)
    padi = jnp.full((EP - E - N,), N, i32)
    src_e = jnp.concatenate([src, loops, padi])
    dst_e = jnp.concatenate([dstm, loops, padi])

    xp = jnp.pad(x.astype(f32), ((0, NP - N), (0, 0)))

    def row(v):
        return v.reshape(1, -1).astype(f32)

    c1 = p['conv1']
    h0, xl, xr = _tc_pre(xp, row(p['g0']), row(p['b0']),
                         c1['Wl'], row(c1['bl']), c1['Wr'], row(c1['br']))
    acc1 = _sc_gat_kernel(HEADS, HC, 48, 2)(xl, xr, src_e, dst_e,
                                            c1['att'].reshape(-1))

    c2 = p['conv2']
    h1, xl, xr = _tc_mid(acc1, row(c1['bias']), row(p['g1']), row(p['b1']),
                         h0, c2['Wl'], row(c2['bl']), c2['Wr'], row(c2['br']), D)
    acc2 = _sc_gat_kernel(HEADS, HC, 48, 2)(xl, xr, src_e, dst_e,
                                            c2['att'].reshape(-1))

    c3 = p['conv3']
    h2, xl, xr = _tc_mid(acc2, row(c2['bias']), row(p['g2']), row(p['b2']),
                         h1, c3['Wl'], row(c3['bl']), c3['Wr'], row(c3['br']), OUT)
    acc3 = _sc_gat_kernel(1, OUT, 80, 4)(xl, xr, src_e, dst_e,
                                          c3['att'].reshape(-1))

    y = _tc_post(acc3, row(c3['bias']), p['Wc1'], row(p['bc1']),
                 p['Wc2'], row(p['bc2']))
    return y[:N]


# speedup vs baseline: 69.2399x; 1.0020x over previous
import functools

import jax
import jax.numpy as jnp
from jax import lax
from jax.experimental import pallas as pl
from jax.experimental.pallas import tpu as pltpu
from jax.experimental.pallas import tpu_sc as plsc

N = 10000
E = 320000
D = 128
HEADS, HC, OUT, CLS_HID = 8, 16, 64, 16

NP = 10112
EP = 330240
NC, NS = 2, 16
NW = NC * NS
EW = EP // NW
RPS = NP // NS

_BN_SCALE = 1.0 / (1.0 + 1e-5) ** 0.5

f32 = jnp.float32
i32 = jnp.int32


@functools.cache
def _sc_gat_kernel(heads, hc, G, UNROLL):
    CHUNKS = EW // G
    hwc = heads * hc
    aw = hwc + 16
    nv = hwc // 16
    vph = hc // 16
    mesh = plsc.VectorSubcoreMesh(core_axis_name="c", subcore_axis_name="s",
                                  num_cores=NC, num_subcores=NS)

    @functools.partial(
        pl.kernel,
        out_type=jax.ShapeDtypeStruct((NC, NP, aw), f32),
        mesh=mesh,
        scratch_types=[
            pltpu.VMEM((G,), i32), pltpu.VMEM((G,), i32),
            pltpu.VMEM((G,), i32), pltpu.VMEM((G,), i32),
            pltpu.VMEM((G,), i32), pltpu.VMEM((G,), i32),
            pltpu.VMEM((G, hwc), f32), pltpu.VMEM((G, hwc), f32),
            pltpu.VMEM((G, hwc), f32), pltpu.VMEM((G, hwc), f32),
            pltpu.VMEM((G, aw), f32), pltpu.VMEM((G, aw), f32),
            pltpu.VMEM((hwc,), f32),
            pltpu.VMEM_SHARED((NP, aw), f32),
            pltpu.SemaphoreType.DMA, pltpu.SemaphoreType.DMA,
            pltpu.SemaphoreType.DMA, pltpu.SemaphoreType.DMA,
            pltpu.SemaphoreType.DMA, pltpu.SemaphoreType.DMA,
            pltpu.SemaphoreType.DMA, pltpu.SemaphoreType.DMA,
        ],
        compiler_params=pltpu.CompilerParams(needs_layout_passes=False,
                                             use_tc_tiling_on_sc=False),
    )
    def kern(xl_hbm, xr_hbm, src_hbm, dst_hbm, att_hbm, out_hbm,
             idxs0, idxs1, idxd0, idxd1, idxc0, idxc1,
             bufl0, bufl1, bufr0, bufr1, val0, val1, att_v, acc,
             seml0, seml1, semr0, semr1, semc0, semc1, semi0, semi1):
        idxs = (idxs0, idxs1)
        idxd = (idxd0, idxd1)
        idxc = (idxc0, idxc1)
        bufl = (bufl0, bufl1)
        bufr = (bufr0, bufr1)
        val = (val0, val1)
        seml = (seml0, seml1)
        semr = (semr0, semr1)
        semc = (semc0, semc1)
        semi = (semi0, semi1)
        c = lax.axis_index("c")
        s = lax.axis_index("s")
        wid = c * NS + s
        lanes = lax.iota(i32, 16)
        zero16 = jnp.zeros((16,), f32)

        pltpu.sync_copy(att_hbm, att_v)
        att_regs = [att_v[pl.ds(16 * j, 16)] for j in range(nv)]

        def zrow(e, _):
            for j in range(aw // 16):
                val0[e, pl.ds(16 * j, 16)] = zero16
            return 0

        lax.fori_loop(i32(0), i32(G), zrow, 0)
        row0 = s * i32(RPS)
        off = 0
        for sz in (G,) * (RPS // G) + ((RPS % G,) if RPS % G else ()):
            pltpu.sync_copy(val0.at[pl.ds(0, sz)], acc.at[pl.ds(row0 + off, sz)])
            off += sz
        plsc.subcore_barrier()

        lane15 = jnp.full((16, 1), 15, i32)
        _gdn = lax.GatherDimensionNumbers(
            offset_dims=(), collapsed_slice_dims=(0,), start_index_map=(0,))

        def _bcast(v, idx_vec):
            return lax.gather(v, idx_vec, _gdn, (1,),
                              mode=lax.GatherScatterMode.PROMISE_IN_BOUNDS)

        def edge_body(bl, br, vl, e):
            zls = []
            ts = []
            for h in range(heads):
                acc_t = None
                for v in range(vph):
                    j = h * vph + v
                    zl = bl[e, pl.ds(16 * j, 16)]
                    zls.append(zl)
                    z = zl + br[e, pl.ds(16 * j, 16)]
                    z = jnp.maximum(z, 0.2 * z)
                    t = z * att_regs[j]
                    acc_t = t if acc_t is None else acc_t + t
                ts.append(acc_t)
            tots = [_bcast(plsc.cumsum(t), lane15) for t in ts]
            alpha = jnp.full((16,), -100.0, f32)
            for h in range(heads):
                alpha = jnp.where(lanes == h, tots[h], alpha)
            wv = jnp.exp(jnp.minimum(alpha, 60.0))
            for h in range(heads):
                wb = _bcast(wv, jnp.full((16, 1), h, i32))
                for v in range(vph):
                    j = h * vph + v
                    vl[e, pl.ds(16 * j, 16)] = zls[j] * wb
            vl[e, pl.ds(hwc, 16)] = wv

        base = wid * i32(EW)
        assert CHUNKS % 2 == 1 and CHUNKS >= 5

        def idx_copy(k, st):
            o = base + k * i32(G)
            pltpu.async_copy(src_hbm.at[pl.ds(o, G)], idxs[st], semi[st])
            pltpu.async_copy(dst_hbm.at[pl.ds(o, G)], idxd[st], semi[st])

        def wait_idx(st):
            pltpu.make_async_copy(src_hbm.at[pl.ds(base, G)], idxs[st],
                                  semi[st]).wait()
            pltpu.make_async_copy(dst_hbm.at[pl.ds(base, G)], idxd[st],
                                  semi[st]).wait()

        def gather(st):
            pltpu.async_copy(xl_hbm.at[idxs[st]], bufl[st], seml[st])
            pltpu.async_copy(xr_hbm.at[idxd[st]], bufr[st], semr[st])

        def wait_gather(st):
            pltpu.make_async_copy(xl_hbm.at[idxs[st]], bufl[st], seml[st]).wait()
            pltpu.make_async_copy(xr_hbm.at[idxd[st]], bufr[st], semr[st]).wait()

        def wait_scatter(st):
            pltpu.make_async_copy(val[st], acc.at[idxc[st]], semc[st]).wait()

        def body(k, k2, st):
            wait_gather(st)
            pl.when(k2 > 0)(lambda: wait_scatter(st))
            for r in range(G // 16):
                idxc[st][pl.ds(16 * r, 16)] = idxd[st][pl.ds(16 * r, 16)]
            pl.when(k + 2 < i32(CHUNKS))(lambda: idx_copy(k + 2, st))

            def launch_next():
                wait_idx(1 - st)
                gather(1 - st)

            pl.when(k + 1 < i32(CHUNKS))(launch_next)

            def edge_group(i, car):
                for u in range(UNROLL):
                    edge_body(bufl[st], bufr[st], val[st], UNROLL * i + u)
                return car

            lax.fori_loop(i32(0), i32(G // UNROLL), edge_group, 0)
            pltpu.async_copy(val[st], acc.at[idxc[st]], semc[st], add=True)

        K2 = CHUNKS // 2
        idx_copy(i32(0), 0)
        idx_copy(i32(1), 1)
        wait_idx(0)
        gather(0)

        def outer(k2, car):
            body(2 * k2, k2, 0)
            body(2 * k2 + 1, k2, 1)
            return car

        lax.fori_loop(i32(0), i32(K2), outer, 0)
        wait_gather(0)
        wait_scatter(0)
        for r in range(G // 16):
            idxc[0][pl.ds(16 * r, 16)] = idxd[0][pl.ds(16 * r, 16)]

        def edge_group_tail(i, car):
            for u in range(UNROLL):
                edge_body(bufl[0], bufr[0], val[0], UNROLL * i + u)
            return car

        lax.fori_loop(i32(0), i32(G // UNROLL), edge_group_tail, 0)
        pltpu.async_copy(val[0], acc.at[idxc[0]], semc[0], add=True)
        wait_scatter(1)
        wait_scatter(0)
        plsc.subcore_barrier()
        plsc.subcore_barrier()

        off = 0
        for sz in (G,) * (RPS // G) + ((RPS % G,) if RPS % G else ()):
            pltpu.sync_copy(acc.at[pl.ds(row0 + off, sz)],
                            out_hbm.at[c, pl.ds(row0 + off, sz)])
            off += sz

    return kern


def _tc_call(body, out_shapes, *args):
    return pl.pallas_call(body, out_shape=out_shapes)(*args)


def _tc_pre(x, g0, b0, wl, bl, wr, br):
    def body(x_ref, g_ref, b_ref, wl_ref, bl_ref, wr_ref, br_ref,
             h_ref, xl_ref, xr_ref):
        h = x_ref[...] * (g_ref[...] * _BN_SCALE) + b_ref[...]
        h_ref[...] = h
        xl_ref[...] = jnp.dot(h, wl_ref[...], preferred_element_type=f32) + bl_ref[...]
        xr_ref[...] = jnp.dot(h, wr_ref[...], preferred_element_type=f32) + br_ref[...]

    outs = [jax.ShapeDtypeStruct((NP, D), f32)] * 3
    return _tc_call(body, outs, x, g0, b0, wl, bl, wr, br)


def _div_den(a, heads, hc):
    hwc = heads * hc
    num = a[:, :hwc]
    den = a[:, hwc:hwc + heads]
    hh = lax.broadcasted_iota(i32, (heads, hwc), 0)
    cc = lax.broadcasted_iota(i32, (heads, hwc), 1)
    rep = jnp.where(cc // hc == hh, f32(1.0), f32(0.0))
    den_rep = jnp.dot(den, rep, preferred_element_type=f32)
    return num / (den_rep + 1e-16)


def _tc_mid(acc, bias, g, b, hprev, wl, bl, wr, br, dout):
    hwc = HEADS * HC

    def body(a_ref, bias_ref, g_ref, b_ref, hp_ref, wl_ref, bl_ref,
             wr_ref, br_ref, h_ref, xl_ref, xr_ref):
        a = a_ref[0] + a_ref[1]
        o = _div_den(a, HEADS, HC) + bias_ref[...]
        o = o * (g_ref[...] * _BN_SCALE) + b_ref[...]
        o = jnp.where(o > 0, o, jnp.exp(o) - 1.0)
        h = o + hp_ref[...]
        h_ref[...] = h
        xl_ref[...] = jnp.dot(h, wl_ref[...], preferred_element_type=f32) + bl_ref[...]
        xr_ref[...] = jnp.dot(h, wr_ref[...], preferred_element_type=f32) + br_ref[...]

    outs = [jax.ShapeDtypeStruct((NP, hwc), f32),
            jax.ShapeDtypeStruct((NP, dout), f32),
            jax.ShapeDtypeStruct((NP, dout), f32)]
    return _tc_call(body, outs, acc, bias, g, b, hprev, wl, bl, wr, br)


def _tc_post(acc, bias, wc1, bc1, wc2, bc2):
    def body(a_ref, bias_ref, w1_ref, b1_ref, w2_ref, b2_ref, y_ref):
        a = a_ref[0] + a_ref[1]
        o = _div_den(a, 1, OUT) + bias_ref[...]
        y = jnp.dot(o, w1_ref[...], preferred_element_type=f32) + b1_ref[...]
        y = jnp.maximum(y, 0.0)
        y_ref[...] = jnp.dot(y, w2_ref[...], preferred_element_type=f32) + b2_ref[...]

    outs = jax.ShapeDtypeStruct((NP, OUT), f32)
    return _tc_call(body, outs, acc, bias, wc1, bc1, wc2, bc2)


def kernel(x, edge_index, params):
    p = params
    src = edge_index[0].astype(i32)
    dst = edge_index[1].astype(i32)
    dstm = jnp.where(src == dst, N, dst)
    loops = jnp.arange(N, dtype=i32)
    padi = jnp.full((EP - E - N,), N, i32)
    src_e = jnp.concatenate([src, loops, padi])
    dst_e = jnp.concatenate([dstm, loops, padi])

    xp = jnp.pad(x.astype(f32), ((0, NP - N), (0, 0)))

    def row(v):
        return v.reshape(1, -1).astype(f32)

    c1 = p['conv1']
    h0, xl, xr = _tc_pre(xp, row(p['g0']), row(p['b0']),
                         c1['Wl'], row(c1['bl']), c1['Wr'], row(c1['br']))
    acc1 = _sc_gat_kernel(HEADS, HC, 48, 2)(xl, xr, src_e, dst_e,
                                            c1['att'].reshape(-1))

    c2 = p['conv2']
    h1, xl, xr = _tc_mid(acc1, row(c1['bias']), row(p['g1']), row(p['b1']),
                         h0, c2['Wl'], row(c2['bl']), c2['Wr'], row(c2['br']), D)
    acc2 = _sc_gat_kernel(HEADS, HC, 48, 2)(xl, xr, src_e, dst_e,
                                            c2['att'].reshape(-1))

    c3 = p['conv3']
    h2, xl, xr = _tc_mid(acc2, row(c2['bias']), row(p['g2']), row(p['b2']),
                         h1, c3['Wl'], row(c3['bl']), c3['Wr'], row(c3['br']), OUT)
    acc3 = _sc_gat_kernel(1, OUT, 80, 8)(xl, xr, src_e, dst_e,
                                          c3['att'].reshape(-1))

    y = _tc_post(acc3, row(c3['bias']), p['Wc1'], row(p['bc1']),
                 p['Wc2'], row(p['bc2']))
    return y[:N]
